# baseline (speedup 1.0000x reference)
import functools

import jax
import jax.numpy as jnp
from jax import lax
from jax.experimental import pallas as pl
from jax.experimental.pallas import tpu as pltpu
from jax.experimental.pallas import tpu_sc as plsc

NN = 10000
NE = 320000
DL = 128

NC = 2
NS = 16
NW = NC * NS
EPT = NE // NW
CHUNK = 80
NCH = EPT // CHUNK
NPS = 624
NPS_LAST = NN - (NS - 1) * NPS

_mesh = plsc.VectorSubcoreMesh(core_axis_name="c", subcore_axis_name="s")


def _mlp_ln_body(x_ref, w1_ref, b1_ref, w2_ref, b2_ref, sc_ref, of_ref, o_ref):
    h = jnp.dot(x_ref[...], w1_ref[...], preferred_element_type=jnp.float32)
    h = jnp.maximum(h + b1_ref[...], 0.0)
    u = jnp.dot(h, w2_ref[...], preferred_element_type=jnp.float32) + b2_ref[...]
    mu = jnp.mean(u, axis=-1, keepdims=True)
    var = jnp.mean((u - mu) ** 2, axis=-1, keepdims=True)
    o_ref[...] = ((u - mu) * lax.rsqrt(var + 1e-5)) * sc_ref[...] + of_ref[...]


def _encode(x, p, tile):
    n, d = x.shape
    w1 = p['W1']
    full = lambda shape: pl.BlockSpec(shape, lambda i: (0, 0))
    return pl.pallas_call(
        _mlp_ln_body,
        grid=(n // tile,),
        in_specs=[
            pl.BlockSpec((tile, d), lambda i: (i, 0)),
            full((d, DL)), full((1, DL)), full((DL, DL)),
            full((1, DL)), full((1, DL)), full((1, DL)),
        ],
        out_specs=pl.BlockSpec((tile, DL), lambda i: (i, 0)),
        out_shape=jax.ShapeDtypeStruct((n, DL), jnp.float32),
    )(x, w1, p['b1'].reshape(1, -1), p['W2'], p['b2'].reshape(1, -1),
      p['scale'].reshape(1, -1), p['offset'].reshape(1, -1))


def _pq_body(x_ref, ws_ref, wr_ref, p_ref, q_ref):
    x = x_ref[...]
    p_ref[...] = jnp.dot(x, ws_ref[...], preferred_element_type=jnp.float32)
    q_ref[...] = jnp.dot(x, wr_ref[...], preferred_element_type=jnp.float32)


def _pq(node_lat, w1s, w1r, tile=1000):
    full = lambda shape: pl.BlockSpec(shape, lambda i: (0, 0))
    row = pl.BlockSpec((tile, DL), lambda i: (i, 0))
    return pl.pallas_call(
        _pq_body,
        grid=(NN // tile,),
        in_specs=[row, full((DL, DL)), full((DL, DL))],
        out_specs=[row, row],
        out_shape=[jax.ShapeDtypeStruct((NN, DL), jnp.float32)] * 2,
    )(node_lat, w1s, w1r)


def _prep_body(g_ref, wge_ref, b1e_ref, wgn_ref, b1n_ref, ce_ref, cn_ref):
    g = g_ref[...]
    ce_ref[...] = jnp.dot(g, wge_ref[...], preferred_element_type=jnp.float32) + b1e_ref[...]
    cn_ref[...] = jnp.dot(g, wgn_ref[...], preferred_element_type=jnp.float32) + b1n_ref[...]


def _prep(glob, wge, b1e, wgn, b1n):
    full = lambda shape: pl.BlockSpec(shape, lambda: (0, 0))
    return pl.pallas_call(
        _prep_body,
        in_specs=[full((1, DL)), full((DL, DL)), full((1, DL)),
                  full((DL, DL)), full((1, DL))],
        out_specs=[full((1, DL)), full((1, DL))],
        out_shape=[jax.ShapeDtypeStruct((1, DL), jnp.float32)] * 2,
    )(glob, wge, b1e.reshape(1, -1), wgn, b1n.reshape(1, -1))


def _edge_body(e_ref, g_ref, w1_ref, c_ref, w2_ref, b2_ref,
               sc_ref, of_ref, o_ref, sum_ref):
    e = e_ref[...]
    h = jnp.dot(e, w1_ref[...], preferred_element_type=jnp.float32)
    h = jnp.maximum(h + g_ref[...] + c_ref[...], 0.0)
    u = jnp.dot(h, w2_ref[...], preferred_element_type=jnp.float32) + b2_ref[...]
    mu = jnp.mean(u, axis=-1, keepdims=True)
    var = jnp.mean((u - mu) ** 2, axis=-1, keepdims=True)
    new = e + ((u - mu) * lax.rsqrt(var + 1e-5)) * sc_ref[...] + of_ref[...]
    o_ref[...] = new

    @pl.when(pl.program_id(0) == 0)
    def _():
        sum_ref[...] = jnp.zeros_like(sum_ref)

    sum_ref[...] += jnp.sum(new, axis=0, keepdims=True)


def _edge_step(edge_lat, g, w1e, ce, p, tile=2000):
    full = lambda shape: pl.BlockSpec(shape, lambda i: (0, 0))
    row = pl.BlockSpec((tile, DL), lambda i: (i, 0))
    return pl.pallas_call(
        _edge_body,
        grid=(NE // tile,),
        in_specs=[row, row, full((DL, DL)), full((1, DL)),
                  full((DL, DL)), full((1, DL)), full((1, DL)), full((1, DL))],
        out_specs=[row, full((1, DL))],
        out_shape=[jax.ShapeDtypeStruct((NE, DL), jnp.float32),
                   jax.ShapeDtypeStruct((1, DL), jnp.float32)],
    )(edge_lat, g, w1e, ce, p['W2'], p['b2'].reshape(1, -1),
      p['scale'].reshape(1, -1), p['offset'].reshape(1, -1))


def _node_body(nl_ref, agg_ref, w1n_ref, w1a_ref, c_ref, w2_ref, b2_ref,
               sc_ref, of_ref, o_ref, sum_ref):
    nl = nl_ref[...]
    agg = agg_ref[0] + agg_ref[1]
    h = jnp.dot(nl, w1n_ref[...], preferred_element_type=jnp.float32)
    h = h + jnp.dot(agg, w1a_ref[...], preferred_element_type=jnp.float32)
    h = jnp.maximum(h + c_ref[...], 0.0)
    u = jnp.dot(h, w2_ref[...], preferred_element_type=jnp.float32) + b2_ref[...]
    mu = jnp.mean(u, axis=-1, keepdims=True)
    var = jnp.mean((u - mu) ** 2, axis=-1, keepdims=True)
    new = nl + ((u - mu) * lax.rsqrt(var + 1e-5)) * sc_ref[...] + of_ref[...]
    o_ref[...] = new

    @pl.when(pl.program_id(0) == 0)
    def _():
        sum_ref[...] = jnp.zeros_like(sum_ref)

    sum_ref[...] += jnp.sum(new, axis=0, keepdims=True)


def _node_step(node_lat, agg2, w1n, w1a, cn, p, tile=1000):
    full = lambda shape: pl.BlockSpec(shape, lambda i: (0, 0))
    row = pl.BlockSpec((tile, DL), lambda i: (i, 0))
    return pl.pallas_call(
        _node_body,
        grid=(NN // tile,),
        in_specs=[row, pl.BlockSpec((2, tile, DL), lambda i: (0, i, 0)),
                  full((DL, DL)), full((DL, DL)), full((1, DL)),
                  full((DL, DL)), full((1, DL)), full((1, DL)), full((1, DL))],
        out_specs=[row, full((1, DL))],
        out_shape=[jax.ShapeDtypeStruct((NN, DL), jnp.float32),
                   jax.ShapeDtypeStruct((1, DL), jnp.float32)],
    )(node_lat, agg2, w1n, w1a, cn, p['W2'], p['b2'].reshape(1, -1),
      p['scale'].reshape(1, -1), p['offset'].reshape(1, -1))


def _glob_body(ns_ref, es_ref, g_ref, wgn_ref, wge_ref, wgg_ref, b1_ref,
               w2_ref, b2_ref, sc_ref, of_ref, o_ref):
    g = g_ref[...]
    h = jnp.dot(ns_ref[...], wgn_ref[...], preferred_element_type=jnp.float32)
    h = h + jnp.dot(es_ref[...], wge_ref[...], preferred_element_type=jnp.float32)
    h = h + jnp.dot(g, wgg_ref[...], preferred_element_type=jnp.float32)
    h = jnp.maximum(h + b1_ref[...], 0.0)
    u = jnp.dot(h, w2_ref[...], preferred_element_type=jnp.float32) + b2_ref[...]
    mu = jnp.mean(u, axis=-1, keepdims=True)
    var = jnp.mean((u - mu) ** 2, axis=-1, keepdims=True)
    o_ref[...] = g + ((u - mu) * lax.rsqrt(var + 1e-5)) * sc_ref[...] + of_ref[...]


def _glob_update(nsum, esum, glob, wgn, wge, wgg, p):
    full = lambda shape: pl.BlockSpec(shape, lambda: (0, 0))
    return pl.pallas_call(
        _glob_body,
        in_specs=[full((1, DL))] * 3 + [full((DL, DL))] * 3 + [full((1, DL)),
                  full((DL, DL)), full((1, DL)), full((1, DL)), full((1, DL))],
        out_specs=full((1, DL)),
        out_shape=jax.ShapeDtypeStruct((1, DL), jnp.float32),
    )(nsum, esum, glob, wgn, wge, wgg, p['b1'].reshape(1, -1), p['W2'],
      p['b2'].reshape(1, -1), p['scale'].reshape(1, -1), p['offset'].reshape(1, -1))


def _decode_body(g_ref, w1_ref, b1_ref, w2_ref, b2_ref, o_ref):
    h = jnp.dot(g_ref[...], w1_ref[...], preferred_element_type=jnp.float32)
    h = jnp.maximum(h + b1_ref[...], 0.0)
    o_ref[...] = jnp.dot(h, w2_ref[...], preferred_element_type=jnp.float32) + b2_ref[...]


def _decode(glob, p):
    full = lambda shape: pl.BlockSpec(shape, lambda: (0, 0))
    return pl.pallas_call(
        _decode_body,
        in_specs=[full((1, DL)), full((DL, DL)), full((1, DL)),
                  full((DL, 1)), full((1, 1))],
        out_specs=full((1, 1)),
        out_shape=jax.ShapeDtypeStruct((1, 1), jnp.float32),
    )(glob, p['W1'], p['b1'].reshape(1, -1), p['W2'], p['b2'].reshape(1, -1))


NBUF = 4


@functools.partial(
    pl.kernel,
    mesh=_mesh,
    out_type=jax.ShapeDtypeStruct((NE, DL), jnp.float32),
    scratch_types=[
        pltpu.VMEM((NCH, CHUNK), jnp.int32),
        pltpu.VMEM((NCH, CHUNK), jnp.int32),
        pltpu.VMEM((NBUF, CHUNK, DL), jnp.float32),
        pltpu.VMEM((NBUF, CHUNK, DL), jnp.float32),
        pltpu.SemaphoreType.DMA,
        pltpu.SemaphoreType.DMA,
    ],
)
def _sc_gather(p_hbm, q_hbm, sidx_hbm, ridx_hbm, g_hbm,
               sidx_v, ridx_v, bp, bq, sg, sw):
    wid = lax.axis_index("s") * NC + lax.axis_index("c")
    base = wid * EPT
    pltpu.sync_copy(sidx_hbm.at[wid], sidx_v)
    pltpu.sync_copy(ridx_hbm.at[wid], ridx_v)

    def start_gather(j, b):
        pltpu.async_copy(p_hbm.at[sidx_v.at[j]], bp.at[b], sg)
        pltpu.async_copy(q_hbm.at[ridx_v.at[j]], bq.at[b], sg)

    def wait_gather(j, b):
        pltpu.make_async_copy(p_hbm.at[sidx_v.at[j]], bp.at[b], sg).wait()
        pltpu.make_async_copy(q_hbm.at[ridx_v.at[j]], bq.at[b], sg).wait()

    def start_write(j, b):
        pltpu.async_copy(bp.at[b], g_hbm.at[pl.ds(base + j * CHUNK, CHUNK)], sw)

    def wait_write(j, b):
        pltpu.make_async_copy(
            bp.at[b], g_hbm.at[pl.ds(base + j * CHUNK, CHUNK)], sw).wait()

    for k in range(NBUF - 1):
        start_gather(k, k)

    def body(j, carry):
        b = lax.rem(j, NBUF)

        @pl.when(j >= 1)
        def _():
            wait_write(j - 1, lax.rem(j - 1, NBUF))

        @pl.when(j + NBUF - 1 < NCH)
        def _():
            start_gather(j + NBUF - 1, lax.rem(j + NBUF - 1, NBUF))

        wait_gather(j, b)

        def add_row(r, c2):
            for l in range(DL // 16):
                sl = pl.ds(l * 16, 16)
                bp[b, r, sl] = bp[b, r, sl] + bq[b, r, sl]
            return c2

        lax.fori_loop(0, CHUNK, add_row, 0)
        start_write(j, b)
        return carry

    lax.fori_loop(0, NCH, body, 0)
    wait_write(NCH - 1, (NCH - 1) % NBUF)


@functools.partial(
    pl.kernel,
    mesh=_mesh,
    out_type=jax.ShapeDtypeStruct((NC, NN, DL), jnp.float32),
    scratch_types=[
        pltpu.VMEM((NCH, CHUNK), jnp.int32),
        pltpu.VMEM((2, CHUNK, DL), jnp.float32),
        pltpu.VMEM_SHARED((NN, DL), jnp.float32),
        pltpu.SemaphoreType.DMA,
    ],
)
def _sc_scatter(e_hbm, ridx_hbm, zeros_hbm, out_hbm, ridx_v, rows_v, agg_sh, sr):
    cid = lax.axis_index("c")
    sid = lax.axis_index("s")
    wid = sid * NC + cid
    base = wid * EPT
    pltpu.sync_copy(ridx_hbm.at[wid], ridx_v)

    @pl.when(sid < NS - 1)
    def _():
        pltpu.sync_copy(zeros_hbm.at[pl.ds(0, NPS)],
                        agg_sh.at[pl.ds(sid * NPS, NPS)])

    @pl.when(sid == NS - 1)
    def _():
        pltpu.sync_copy(zeros_hbm, agg_sh.at[pl.ds((NS - 1) * NPS, NPS_LAST)])

    plsc.subcore_barrier()

    def start_read(j, b):
        pltpu.async_copy(e_hbm.at[pl.ds(base + j * CHUNK, CHUNK)],
                         rows_v.at[b], sr)

    def wait_read(j, b):
        pltpu.make_async_copy(e_hbm.at[pl.ds(base + j * CHUNK, CHUNK)],
                              rows_v.at[b], sr).wait()

    start_read(0, 0)

    def body(j, carry):
        b = lax.rem(j, 2)

        @pl.when(j + 1 < NCH)
        def _():
            start_read(j + 1, 1 - b)

        wait_read(j, b)
        pltpu.sync_copy(rows_v.at[b], agg_sh.at[ridx_v.at[j]], add=True)
        return carry

    lax.fori_loop(0, NCH, body, 0)
    plsc.subcore_barrier()

    @pl.when(sid < NS - 1)
    def _():
        pltpu.sync_copy(agg_sh.at[pl.ds(sid * NPS, NPS)],
                        out_hbm.at[cid, pl.ds(sid * NPS, NPS)])

    @pl.when(sid == NS - 1)
    def _():
        pltpu.sync_copy(agg_sh.at[pl.ds((NS - 1) * NPS, NPS_LAST)],
                        out_hbm.at[cid, pl.ds((NS - 1) * NPS, NPS_LAST)])


def kernel(nodes, edges, senders, receivers, params):
    nodes_p = jnp.pad(nodes, ((0, 0), (0, 3)))
    edges_p = jnp.pad(edges, ((0, 0), (0, 3)))
    pne = dict(params['node_enc'])
    pne['W1'] = jnp.pad(params['node_enc']['W1'], ((0, 3), (0, 0)))
    pee = dict(params['edge_enc'])
    pee['W1'] = jnp.pad(params['edge_enc']['W1'], ((0, 3), (0, 0)))

    pe = params['edge_mlp']
    w1e, w1s, w1r, w1ge = (pe['W1'][0:128], pe['W1'][128:256],
                           pe['W1'][256:384], pe['W1'][384:512])
    pn = params['node_mlp']
    w1n, w1a, w1gn = pn['W1'][0:128], pn['W1'][128:256], pn['W1'][256:384]
    pg = params['glob_mlp']
    wgn, wge, wgg = pg['W1'][0:128], pg['W1'][128:256], pg['W1'][256:384]

    sidx = senders.reshape(NW, NCH, CHUNK)
    ridx = receivers.reshape(NW, NCH, CHUNK)
    zeros_slice = jnp.zeros((NPS_LAST, DL), jnp.float32)

    node_lat = _encode(nodes_p, pne, tile=1000)
    edge_lat = _encode(edges_p, pee, tile=2000)
    glob = jnp.zeros((1, DL), jnp.float32)

    for _ in range(4):
        ce, cn = _prep(glob, w1ge, pe['b1'], w1gn, pn['b1'])
        pt, qt = _pq(node_lat, w1s, w1r)
        g = _sc_gather(pt, qt, sidx, ridx)
        edge_lat, esum = _edge_step(edge_lat, g, w1e, ce, pe)
        agg2 = _sc_scatter(edge_lat, ridx, zeros_slice)
        node_lat, nsum = _node_step(node_lat, agg2, w1n, w1a, cn, pn)
        glob = _glob_update(nsum, esum, glob, wgn, wge, wgg, pg)

    out = _decode(glob, params['decoder'])
    return out * 1.0 + 0.0


# speedup vs baseline: 1.0595x; 1.0595x over previous
import functools

import jax
import jax.numpy as jnp
from jax import lax
from jax.experimental import pallas as pl
from jax.experimental.pallas import tpu as pltpu
from jax.experimental.pallas import tpu_sc as plsc

NN = 10000
NE = 320000
DL = 128

NC = 2
NS = 16
NW = NC * NS
EPT = NE // NW
CHUNK = 80
NCH = EPT // CHUNK
NPS = 624
NPS_LAST = NN - (NS - 1) * NPS

_mesh = plsc.VectorSubcoreMesh(core_axis_name="c", subcore_axis_name="s")


def _mlp_ln_body(x_ref, w1_ref, b1_ref, w2_ref, b2_ref, sc_ref, of_ref, o_ref):
    h = jnp.dot(x_ref[...], w1_ref[...], preferred_element_type=jnp.float32)
    h = jnp.maximum(h + b1_ref[...], 0.0)
    u = jnp.dot(h, w2_ref[...], preferred_element_type=jnp.float32) + b2_ref[...]
    mu = jnp.mean(u, axis=-1, keepdims=True)
    var = jnp.mean((u - mu) ** 2, axis=-1, keepdims=True)
    o_ref[...] = ((u - mu) * lax.rsqrt(var + 1e-5)) * sc_ref[...] + of_ref[...]


def _encode(x, p, tile):
    n, d = x.shape
    w1 = p['W1']
    full = lambda shape: pl.BlockSpec(shape, lambda i: (0, 0))
    return pl.pallas_call(
        _mlp_ln_body,
        grid=(n // tile,),
        in_specs=[
            pl.BlockSpec((tile, d), lambda i: (i, 0)),
            full((d, DL)), full((1, DL)), full((DL, DL)),
            full((1, DL)), full((1, DL)), full((1, DL)),
        ],
        out_specs=pl.BlockSpec((tile, DL), lambda i: (i, 0)),
        out_shape=jax.ShapeDtypeStruct((n, DL), jnp.float32),
    )(x, w1, p['b1'].reshape(1, -1), p['W2'], p['b2'].reshape(1, -1),
      p['scale'].reshape(1, -1), p['offset'].reshape(1, -1))


def _pq_body(x_ref, ws_ref, wr_ref, p_ref, q_ref):
    x = x_ref[...]
    p_ref[...] = jnp.dot(x, ws_ref[...], preferred_element_type=jnp.float32)
    q_ref[...] = jnp.dot(x, wr_ref[...], preferred_element_type=jnp.float32)


def _pq(node_lat, w1s, w1r, tile=1000):
    full = lambda shape: pl.BlockSpec(shape, lambda i: (0, 0))
    row = pl.BlockSpec((tile, DL), lambda i: (i, 0))
    return pl.pallas_call(
        _pq_body,
        grid=(NN // tile,),
        in_specs=[row, full((DL, DL)), full((DL, DL))],
        out_specs=[row, row],
        out_shape=[jax.ShapeDtypeStruct((NN, DL), jnp.float32)] * 2,
    )(node_lat, w1s, w1r)


def _prep_body(g_ref, wge_ref, b1e_ref, wgn_ref, b1n_ref, ce_ref, cn_ref):
    g = g_ref[...]
    ce_ref[...] = jnp.dot(g, wge_ref[...], preferred_element_type=jnp.float32) + b1e_ref[...]
    cn_ref[...] = jnp.dot(g, wgn_ref[...], preferred_element_type=jnp.float32) + b1n_ref[...]


def _prep(glob, wge, b1e, wgn, b1n):
    full = lambda shape: pl.BlockSpec(shape, lambda: (0, 0))
    return pl.pallas_call(
        _prep_body,
        in_specs=[full((1, DL)), full((DL, DL)), full((1, DL)),
                  full((DL, DL)), full((1, DL))],
        out_specs=[full((1, DL)), full((1, DL))],
        out_shape=[jax.ShapeDtypeStruct((1, DL), jnp.float32)] * 2,
    )(glob, wge, b1e.reshape(1, -1), wgn, b1n.reshape(1, -1))


def _edge_body(e_ref, g_ref, w1_ref, c_ref, w2_ref, b2_ref,
               sc_ref, of_ref, o_ref, sum_ref):
    e = e_ref[...]
    h = jnp.dot(e, w1_ref[...], preferred_element_type=jnp.float32)
    h = jnp.maximum(h + g_ref[...] + c_ref[...], 0.0)
    u = jnp.dot(h, w2_ref[...], preferred_element_type=jnp.float32) + b2_ref[...]
    mu = jnp.mean(u, axis=-1, keepdims=True)
    var = jnp.mean((u - mu) ** 2, axis=-1, keepdims=True)
    new = e + ((u - mu) * lax.rsqrt(var + 1e-5)) * sc_ref[...] + of_ref[...]
    o_ref[...] = new

    @pl.when(pl.program_id(0) == 0)
    def _():
        sum_ref[...] = jnp.zeros_like(sum_ref)

    sum_ref[...] += jnp.sum(new, axis=0, keepdims=True)


def _edge_step(edge_lat, g, w1e, ce, p, tile=2000):
    full = lambda shape: pl.BlockSpec(shape, lambda i: (0, 0))
    row = pl.BlockSpec((tile, DL), lambda i: (i, 0))
    return pl.pallas_call(
        _edge_body,
        grid=(NE // tile,),
        in_specs=[row, row, full((DL, DL)), full((1, DL)),
                  full((DL, DL)), full((1, DL)), full((1, DL)), full((1, DL))],
        out_specs=[row, full((1, DL))],
        out_shape=[jax.ShapeDtypeStruct((NE, DL), jnp.float32),
                   jax.ShapeDtypeStruct((1, DL), jnp.float32)],
    )(edge_lat, g, w1e, ce, p['W2'], p['b2'].reshape(1, -1),
      p['scale'].reshape(1, -1), p['offset'].reshape(1, -1))


def _node_body(nl_ref, agg_ref, w1n_ref, w1a_ref, c_ref, w2_ref, b2_ref,
               sc_ref, of_ref, o_ref, sum_ref):
    nl = nl_ref[...]
    agg = agg_ref[0] + agg_ref[1]
    h = jnp.dot(nl, w1n_ref[...], preferred_element_type=jnp.float32)
    h = h + jnp.dot(agg, w1a_ref[...], preferred_element_type=jnp.float32)
    h = jnp.maximum(h + c_ref[...], 0.0)
    u = jnp.dot(h, w2_ref[...], preferred_element_type=jnp.float32) + b2_ref[...]
    mu = jnp.mean(u, axis=-1, keepdims=True)
    var = jnp.mean((u - mu) ** 2, axis=-1, keepdims=True)
    new = nl + ((u - mu) * lax.rsqrt(var + 1e-5)) * sc_ref[...] + of_ref[...]
    o_ref[...] = new

    @pl.when(pl.program_id(0) == 0)
    def _():
        sum_ref[...] = jnp.zeros_like(sum_ref)

    sum_ref[...] += jnp.sum(new, axis=0, keepdims=True)


def _node_step(node_lat, agg2, w1n, w1a, cn, p, tile=1000):
    full = lambda shape: pl.BlockSpec(shape, lambda i: (0, 0))
    row = pl.BlockSpec((tile, DL), lambda i: (i, 0))
    return pl.pallas_call(
        _node_body,
        grid=(NN // tile,),
        in_specs=[row, pl.BlockSpec((2, tile, DL), lambda i: (0, i, 0)),
                  full((DL, DL)), full((DL, DL)), full((1, DL)),
                  full((DL, DL)), full((1, DL)), full((1, DL)), full((1, DL))],
        out_specs=[row, full((1, DL))],
        out_shape=[jax.ShapeDtypeStruct((NN, DL), jnp.float32),
                   jax.ShapeDtypeStruct((1, DL), jnp.float32)],
    )(node_lat, agg2, w1n, w1a, cn, p['W2'], p['b2'].reshape(1, -1),
      p['scale'].reshape(1, -1), p['offset'].reshape(1, -1))


def _glob_body(ns_ref, es_ref, g_ref, wgn_ref, wge_ref, wgg_ref, b1_ref,
               w2_ref, b2_ref, sc_ref, of_ref, o_ref):
    g = g_ref[...]
    h = jnp.dot(ns_ref[...], wgn_ref[...], preferred_element_type=jnp.float32)
    h = h + jnp.dot(es_ref[...], wge_ref[...], preferred_element_type=jnp.float32)
    h = h + jnp.dot(g, wgg_ref[...], preferred_element_type=jnp.float32)
    h = jnp.maximum(h + b1_ref[...], 0.0)
    u = jnp.dot(h, w2_ref[...], preferred_element_type=jnp.float32) + b2_ref[...]
    mu = jnp.mean(u, axis=-1, keepdims=True)
    var = jnp.mean((u - mu) ** 2, axis=-1, keepdims=True)
    o_ref[...] = g + ((u - mu) * lax.rsqrt(var + 1e-5)) * sc_ref[...] + of_ref[...]


def _glob_update(nsum, esum, glob, wgn, wge, wgg, p):
    full = lambda shape: pl.BlockSpec(shape, lambda: (0, 0))
    return pl.pallas_call(
        _glob_body,
        in_specs=[full((1, DL))] * 3 + [full((DL, DL))] * 3 + [full((1, DL)),
                  full((DL, DL)), full((1, DL)), full((1, DL)), full((1, DL))],
        out_specs=full((1, DL)),
        out_shape=jax.ShapeDtypeStruct((1, DL), jnp.float32),
    )(nsum, esum, glob, wgn, wge, wgg, p['b1'].reshape(1, -1), p['W2'],
      p['b2'].reshape(1, -1), p['scale'].reshape(1, -1), p['offset'].reshape(1, -1))


def _decode_body(g_ref, w1_ref, b1_ref, w2_ref, b2_ref, o_ref):
    h = jnp.dot(g_ref[...], w1_ref[...], preferred_element_type=jnp.float32)
    h = jnp.maximum(h + b1_ref[...], 0.0)
    o_ref[...] = jnp.dot(h, w2_ref[...], preferred_element_type=jnp.float32) + b2_ref[...]


def _decode(glob, p):
    full = lambda shape: pl.BlockSpec(shape, lambda: (0, 0))
    return pl.pallas_call(
        _decode_body,
        in_specs=[full((1, DL)), full((DL, DL)), full((1, DL)),
                  full((DL, 1)), full((1, 1))],
        out_specs=full((1, 1)),
        out_shape=jax.ShapeDtypeStruct((1, 1), jnp.float32),
    )(glob, p['W1'], p['b1'].reshape(1, -1), p['W2'], p['b2'].reshape(1, -1))


NBUF = 4


@functools.partial(
    pl.kernel,
    mesh=_mesh,
    out_type=jax.ShapeDtypeStruct((NE, DL), jnp.float32),
    scratch_types=[
        pltpu.VMEM((NCH, CHUNK), jnp.int32),
        pltpu.VMEM((NCH, CHUNK), jnp.int32),
        pltpu.VMEM((NBUF, CHUNK, DL), jnp.float32),
        pltpu.VMEM((NBUF, CHUNK, DL), jnp.float32),
        pltpu.SemaphoreType.DMA,
        pltpu.SemaphoreType.DMA,
    ],
)
def _sc_gather(p_hbm, q_hbm, sidx_hbm, ridx_hbm, g_hbm,
               sidx_v, ridx_v, bp, bq, sg, sw):
    wid = lax.axis_index("s") * NC + lax.axis_index("c")
    base = wid * EPT
    pltpu.sync_copy(sidx_hbm.at[wid], sidx_v)
    pltpu.sync_copy(ridx_hbm.at[wid], ridx_v)

    def start_gather(j, b):
        pltpu.async_copy(p_hbm.at[sidx_v.at[j]], bp.at[b], sg)
        pltpu.async_copy(q_hbm.at[ridx_v.at[j]], bq.at[b], sg)

    def wait_gather(j, b):
        pltpu.make_async_copy(p_hbm.at[sidx_v.at[j]], bp.at[b], sg).wait()
        pltpu.make_async_copy(q_hbm.at[ridx_v.at[j]], bq.at[b], sg).wait()

    def start_write(j, b):
        pltpu.async_copy(bp.at[b], g_hbm.at[pl.ds(base + j * CHUNK, CHUNK)], sw)

    def wait_write(j, b):
        pltpu.make_async_copy(
            bp.at[b], g_hbm.at[pl.ds(base + j * CHUNK, CHUNK)], sw).wait()

    LOOK = 2
    for k in range(LOOK):
        start_gather(k, k)

    def body(j, carry):
        b = lax.rem(j, NBUF)

        @pl.when(j >= NBUF - LOOK)
        def _():
            wait_write(j - (NBUF - LOOK), lax.rem(j + LOOK, NBUF))

        @pl.when(j + LOOK < NCH)
        def _():
            start_gather(j + LOOK, lax.rem(j + LOOK, NBUF))

        wait_gather(j, b)

        def add_row(r, c2):
            for l in range(DL // 16):
                sl = pl.ds(l * 16, 16)
                bp[b, r, sl] = bp[b, r, sl] + bq[b, r, sl]
            return c2

        lax.fori_loop(0, CHUNK, add_row, 0)
        start_write(j, b)
        return carry

    lax.fori_loop(0, NCH, body, 0)
    for k in range(NBUF - LOOK):
        j = NCH - (NBUF - LOOK) + k
        wait_write(j, j % NBUF)


@functools.partial(
    pl.kernel,
    mesh=_mesh,
    out_type=jax.ShapeDtypeStruct((NC, NN, DL), jnp.float32),
    scratch_types=[
        pltpu.VMEM((NCH, CHUNK), jnp.int32),
        pltpu.VMEM((2, CHUNK, DL), jnp.float32),
        pltpu.VMEM_SHARED((NN, DL), jnp.float32),
        pltpu.SemaphoreType.DMA,
    ],
)
def _sc_scatter(e_hbm, ridx_hbm, zeros_hbm, out_hbm, ridx_v, rows_v, agg_sh, sr):
    cid = lax.axis_index("c")
    sid = lax.axis_index("s")
    wid = sid * NC + cid
    base = wid * EPT
    pltpu.sync_copy(ridx_hbm.at[wid], ridx_v)

    @pl.when(sid < NS - 1)
    def _():
        pltpu.sync_copy(zeros_hbm.at[pl.ds(0, NPS)],
                        agg_sh.at[pl.ds(sid * NPS, NPS)])

    @pl.when(sid == NS - 1)
    def _():
        pltpu.sync_copy(zeros_hbm, agg_sh.at[pl.ds((NS - 1) * NPS, NPS_LAST)])

    plsc.subcore_barrier()

    def start_read(j, b):
        pltpu.async_copy(e_hbm.at[pl.ds(base + j * CHUNK, CHUNK)],
                         rows_v.at[b], sr)

    def wait_read(j, b):
        pltpu.make_async_copy(e_hbm.at[pl.ds(base + j * CHUNK, CHUNK)],
                              rows_v.at[b], sr).wait()

    start_read(0, 0)

    def body(j, carry):
        b = lax.rem(j, 2)

        @pl.when(j + 1 < NCH)
        def _():
            start_read(j + 1, 1 - b)

        wait_read(j, b)
        pltpu.sync_copy(rows_v.at[b], agg_sh.at[ridx_v.at[j]], add=True)
        return carry

    lax.fori_loop(0, NCH, body, 0)
    plsc.subcore_barrier()

    @pl.when(sid < NS - 1)
    def _():
        pltpu.sync_copy(agg_sh.at[pl.ds(sid * NPS, NPS)],
                        out_hbm.at[cid, pl.ds(sid * NPS, NPS)])

    @pl.when(sid == NS - 1)
    def _():
        pltpu.sync_copy(agg_sh.at[pl.ds((NS - 1) * NPS, NPS_LAST)],
                        out_hbm.at[cid, pl.ds((NS - 1) * NPS, NPS_LAST)])


def kernel(nodes, edges, senders, receivers, params):
    nodes_p = jnp.pad(nodes, ((0, 0), (0, 3)))
    edges_p = jnp.pad(edges, ((0, 0), (0, 3)))
    pne = dict(params['node_enc'])
    pne['W1'] = jnp.pad(params['node_enc']['W1'], ((0, 3), (0, 0)))
    pee = dict(params['edge_enc'])
    pee['W1'] = jnp.pad(params['edge_enc']['W1'], ((0, 3), (0, 0)))

    pe = params['edge_mlp']
    w1e, w1s, w1r, w1ge = (pe['W1'][0:128], pe['W1'][128:256],
                           pe['W1'][256:384], pe['W1'][384:512])
    pn = params['node_mlp']
    w1n, w1a, w1gn = pn['W1'][0:128], pn['W1'][128:256], pn['W1'][256:384]
    pg = params['glob_mlp']
    wgn, wge, wgg = pg['W1'][0:128], pg['W1'][128:256], pg['W1'][256:384]

    sidx = senders.reshape(NW, NCH, CHUNK)
    ridx = receivers.reshape(NW, NCH, CHUNK)
    zeros_slice = jnp.zeros((NPS_LAST, DL), jnp.float32)

    node_lat = _encode(nodes_p, pne, tile=1000)
    edge_lat = _encode(edges_p, pee, tile=2000)
    glob = jnp.zeros((1, DL), jnp.float32)

    for _ in range(4):
        ce, cn = _prep(glob, w1ge, pe['b1'], w1gn, pn['b1'])
        pt, qt = _pq(node_lat, w1s, w1r)
        g = _sc_gather(pt, qt, sidx, ridx)
        edge_lat, esum = _edge_step(edge_lat, g, w1e, ce, pe)
        agg2 = _sc_scatter(edge_lat, ridx, zeros_slice)
        node_lat, nsum = _node_step(node_lat, agg2, w1n, w1a, cn, pn)
        glob = _glob_update(nsum, esum, glob, wgn, wge, wgg, pg)

    out = _decode(glob, params['decoder'])
    return out * 1.0 + 0.0


# speedup vs baseline: 1.2590x; 1.1883x over previous
import functools

import jax
import jax.numpy as jnp
from jax import lax
from jax.experimental import pallas as pl
from jax.experimental.pallas import tpu as pltpu
from jax.experimental.pallas import tpu_sc as plsc

NN = 10000
NE = 320000
DL = 128

NC = 2
NS = 16
NW = NC * NS
EPT = NE // NW
CHUNK = 80
NCH = EPT // CHUNK
NPS = 624
NPS_LAST = NN - (NS - 1) * NPS

_mesh = plsc.VectorSubcoreMesh(core_axis_name="c", subcore_axis_name="s")


def _mlp_ln_body(x_ref, w1_ref, b1_ref, w2_ref, b2_ref, sc_ref, of_ref, o_ref):
    h = jnp.dot(x_ref[...], w1_ref[...], preferred_element_type=jnp.float32)
    h = jnp.maximum(h + b1_ref[...], 0.0)
    u = jnp.dot(h, w2_ref[...], preferred_element_type=jnp.float32) + b2_ref[...]
    mu = jnp.mean(u, axis=-1, keepdims=True)
    var = jnp.mean((u - mu) ** 2, axis=-1, keepdims=True)
    o_ref[...] = ((u - mu) * lax.rsqrt(var + 1e-5)) * sc_ref[...] + of_ref[...]


def _encode(x, p, tile):
    n, d = x.shape
    w1 = p['W1']
    full = lambda shape: pl.BlockSpec(shape, lambda i: (0, 0))
    return pl.pallas_call(
        _mlp_ln_body,
        grid=(n // tile,),
        in_specs=[
            pl.BlockSpec((tile, d), lambda i: (i, 0)),
            full((d, DL)), full((1, DL)), full((DL, DL)),
            full((1, DL)), full((1, DL)), full((1, DL)),
        ],
        out_specs=pl.BlockSpec((tile, DL), lambda i: (i, 0)),
        out_shape=jax.ShapeDtypeStruct((n, DL), jnp.float32),
    )(x, w1, p['b1'].reshape(1, -1), p['W2'], p['b2'].reshape(1, -1),
      p['scale'].reshape(1, -1), p['offset'].reshape(1, -1))


def _pq_body(x_ref, ws_ref, wr_ref, p_ref, q_ref):
    x = x_ref[...]
    p_ref[...] = jnp.dot(x, ws_ref[...], preferred_element_type=jnp.float32)
    q_ref[...] = jnp.dot(x, wr_ref[...], preferred_element_type=jnp.float32)


def _pq(node_lat, w1s, w1r, tile=1000):
    full = lambda shape: pl.BlockSpec(shape, lambda i: (0, 0))
    row = pl.BlockSpec((tile, DL), lambda i: (i, 0))
    return pl.pallas_call(
        _pq_body,
        grid=(NN // tile,),
        in_specs=[row, full((DL, DL)), full((DL, DL))],
        out_specs=[row, row],
        out_shape=[jax.ShapeDtypeStruct((NN, DL), jnp.float32)] * 2,
    )(node_lat, w1s, w1r)


def _prep_body(g_ref, wge_ref, b1e_ref, wgn_ref, b1n_ref, ce_ref, cn_ref):
    g = g_ref[...]
    ce_ref[...] = jnp.dot(g, wge_ref[...], preferred_element_type=jnp.float32) + b1e_ref[...]
    cn_ref[...] = jnp.dot(g, wgn_ref[...], preferred_element_type=jnp.float32) + b1n_ref[...]


def _prep(glob, wge, b1e, wgn, b1n):
    full = lambda shape: pl.BlockSpec(shape, lambda: (0, 0))
    return pl.pallas_call(
        _prep_body,
        in_specs=[full((1, DL)), full((DL, DL)), full((1, DL)),
                  full((DL, DL)), full((1, DL))],
        out_specs=[full((1, DL)), full((1, DL))],
        out_shape=[jax.ShapeDtypeStruct((1, DL), jnp.float32)] * 2,
    )(glob, wge, b1e.reshape(1, -1), wgn, b1n.reshape(1, -1))


def _edge_body(e_ref, gp_ref, gq_ref, w1_ref, c_ref, w2_ref, b2_ref,
               sc_ref, of_ref, o_ref, sum_ref):
    e = e_ref[...]
    h = jnp.dot(e, w1_ref[...], preferred_element_type=jnp.float32)
    h = jnp.maximum(h + gp_ref[...] + gq_ref[...] + c_ref[...], 0.0)
    u = jnp.dot(h, w2_ref[...], preferred_element_type=jnp.float32) + b2_ref[...]
    mu = jnp.mean(u, axis=-1, keepdims=True)
    var = jnp.mean((u - mu) ** 2, axis=-1, keepdims=True)
    new = e + ((u - mu) * lax.rsqrt(var + 1e-5)) * sc_ref[...] + of_ref[...]
    o_ref[...] = new

    @pl.when(pl.program_id(0) == 0)
    def _():
        sum_ref[...] = jnp.zeros_like(sum_ref)

    sum_ref[...] += jnp.sum(new, axis=0, keepdims=True)


def _edge_step(edge_lat, gp, gq, w1e, ce, p, tile=2000):
    full = lambda shape: pl.BlockSpec(shape, lambda i: (0, 0))
    row = pl.BlockSpec((tile, DL), lambda i: (i, 0))
    return pl.pallas_call(
        _edge_body,
        grid=(NE // tile,),
        in_specs=[row, row, row, full((DL, DL)), full((1, DL)),
                  full((DL, DL)), full((1, DL)), full((1, DL)), full((1, DL))],
        out_specs=[row, full((1, DL))],
        out_shape=[jax.ShapeDtypeStruct((NE, DL), jnp.float32),
                   jax.ShapeDtypeStruct((1, DL), jnp.float32)],
    )(edge_lat, gp, gq, w1e, ce, p['W2'], p['b2'].reshape(1, -1),
      p['scale'].reshape(1, -1), p['offset'].reshape(1, -1))


def _node_body(nl_ref, agg_ref, w1n_ref, w1a_ref, c_ref, w2_ref, b2_ref,
               sc_ref, of_ref, o_ref, sum_ref):
    nl = nl_ref[...]
    agg = agg_ref[0] + agg_ref[1]
    h = jnp.dot(nl, w1n_ref[...], preferred_element_type=jnp.float32)
    h = h + jnp.dot(agg, w1a_ref[...], preferred_element_type=jnp.float32)
    h = jnp.maximum(h + c_ref[...], 0.0)
    u = jnp.dot(h, w2_ref[...], preferred_element_type=jnp.float32) + b2_ref[...]
    mu = jnp.mean(u, axis=-1, keepdims=True)
    var = jnp.mean((u - mu) ** 2, axis=-1, keepdims=True)
    new = nl + ((u - mu) * lax.rsqrt(var + 1e-5)) * sc_ref[...] + of_ref[...]
    o_ref[...] = new

    @pl.when(pl.program_id(0) == 0)
    def _():
        sum_ref[...] = jnp.zeros_like(sum_ref)

    sum_ref[...] += jnp.sum(new, axis=0, keepdims=True)


def _node_step(node_lat, agg2, w1n, w1a, cn, p, tile=1000):
    full = lambda shape: pl.BlockSpec(shape, lambda i: (0, 0))
    row = pl.BlockSpec((tile, DL), lambda i: (i, 0))
    return pl.pallas_call(
        _node_body,
        grid=(NN // tile,),
        in_specs=[row, pl.BlockSpec((2, tile, DL), lambda i: (0, i, 0)),
                  full((DL, DL)), full((DL, DL)), full((1, DL)),
                  full((DL, DL)), full((1, DL)), full((1, DL)), full((1, DL))],
        out_specs=[row, full((1, DL))],
        out_shape=[jax.ShapeDtypeStruct((NN, DL), jnp.float32),
                   jax.ShapeDtypeStruct((1, DL), jnp.float32)],
    )(node_lat, agg2, w1n, w1a, cn, p['W2'], p['b2'].reshape(1, -1),
      p['scale'].reshape(1, -1), p['offset'].reshape(1, -1))


def _glob_body(ns_ref, es_ref, g_ref, wgn_ref, wge_ref, wgg_ref, b1_ref,
               w2_ref, b2_ref, sc_ref, of_ref, o_ref):
    g = g_ref[...]
    h = jnp.dot(ns_ref[...], wgn_ref[...], preferred_element_type=jnp.float32)
    h = h + jnp.dot(es_ref[...], wge_ref[...], preferred_element_type=jnp.float32)
    h = h + jnp.dot(g, wgg_ref[...], preferred_element_type=jnp.float32)
    h = jnp.maximum(h + b1_ref[...], 0.0)
    u = jnp.dot(h, w2_ref[...], preferred_element_type=jnp.float32) + b2_ref[...]
    mu = jnp.mean(u, axis=-1, keepdims=True)
    var = jnp.mean((u - mu) ** 2, axis=-1, keepdims=True)
    o_ref[...] = g + ((u - mu) * lax.rsqrt(var + 1e-5)) * sc_ref[...] + of_ref[...]


def _glob_update(nsum, esum, glob, wgn, wge, wgg, p):
    full = lambda shape: pl.BlockSpec(shape, lambda: (0, 0))
    return pl.pallas_call(
        _glob_body,
        in_specs=[full((1, DL))] * 3 + [full((DL, DL))] * 3 + [full((1, DL)),
                  full((DL, DL)), full((1, DL)), full((1, DL)), full((1, DL))],
        out_specs=full((1, DL)),
        out_shape=jax.ShapeDtypeStruct((1, DL), jnp.float32),
    )(nsum, esum, glob, wgn, wge, wgg, p['b1'].reshape(1, -1), p['W2'],
      p['b2'].reshape(1, -1), p['scale'].reshape(1, -1), p['offset'].reshape(1, -1))


def _decode_body(g_ref, w1_ref, b1_ref, w2_ref, b2_ref, o_ref):
    h = jnp.dot(g_ref[...], w1_ref[...], preferred_element_type=jnp.float32)
    h = jnp.maximum(h + b1_ref[...], 0.0)
    o_ref[...] = jnp.dot(h, w2_ref[...], preferred_element_type=jnp.float32) + b2_ref[...]


def _decode(glob, p):
    full = lambda shape: pl.BlockSpec(shape, lambda: (0, 0))
    return pl.pallas_call(
        _decode_body,
        in_specs=[full((1, DL)), full((DL, DL)), full((1, DL)),
                  full((DL, 1)), full((1, 1))],
        out_specs=full((1, 1)),
        out_shape=jax.ShapeDtypeStruct((1, 1), jnp.float32),
    )(glob, p['W1'], p['b1'].reshape(1, -1), p['W2'], p['b2'].reshape(1, -1))


NBUF = 4


@functools.partial(
    pl.kernel,
    mesh=_mesh,
    out_type=[jax.ShapeDtypeStruct((NE, DL), jnp.float32),
              jax.ShapeDtypeStruct((NE, DL), jnp.float32)],
    scratch_types=[
        pltpu.VMEM((NCH, CHUNK), jnp.int32),
        pltpu.VMEM((NCH, CHUNK), jnp.int32),
        pltpu.VMEM((NBUF, CHUNK, DL), jnp.float32),
        pltpu.VMEM((NBUF, CHUNK, DL), jnp.float32),
        pltpu.SemaphoreType.DMA,
        pltpu.SemaphoreType.DMA,
    ],
)
def _sc_gather(p_hbm, q_hbm, sidx_hbm, ridx_hbm, gp_hbm, gq_hbm,
               sidx_v, ridx_v, bp, bq, sg, sw):
    wid = lax.axis_index("s") * NC + lax.axis_index("c")
    base = wid * EPT
    pltpu.sync_copy(sidx_hbm.at[wid], sidx_v)
    pltpu.sync_copy(ridx_hbm.at[wid], ridx_v)

    def start_gather(j, b):
        pltpu.async_copy(p_hbm.at[sidx_v.at[j]], bp.at[b], sg)
        pltpu.async_copy(q_hbm.at[ridx_v.at[j]], bq.at[b], sg)

    def wait_gather(j, b):
        pltpu.make_async_copy(p_hbm.at[sidx_v.at[j]], bp.at[b], sg).wait()
        pltpu.make_async_copy(q_hbm.at[ridx_v.at[j]], bq.at[b], sg).wait()

    def start_write(j, b):
        sl = pl.ds(base + j * CHUNK, CHUNK)
        pltpu.async_copy(bp.at[b], gp_hbm.at[sl], sw)
        pltpu.async_copy(bq.at[b], gq_hbm.at[sl], sw)

    def wait_write(j, b):
        sl = pl.ds(base + j * CHUNK, CHUNK)
        pltpu.make_async_copy(bp.at[b], gp_hbm.at[sl], sw).wait()
        pltpu.make_async_copy(bq.at[b], gq_hbm.at[sl], sw).wait()

    LOOK = 2
    for k in range(LOOK):
        start_gather(k, k)

    def body(j, carry):
        b = lax.rem(j, NBUF)

        @pl.when(j >= NBUF - LOOK)
        def _():
            wait_write(j - (NBUF - LOOK), lax.rem(j + LOOK, NBUF))

        @pl.when(j + LOOK < NCH)
        def _():
            start_gather(j + LOOK, lax.rem(j + LOOK, NBUF))

        wait_gather(j, b)
        start_write(j, b)
        return carry

    lax.fori_loop(0, NCH, body, 0)
    for k in range(NBUF - LOOK):
        j = NCH - (NBUF - LOOK) + k
        wait_write(j, j % NBUF)


@functools.partial(
    pl.kernel,
    mesh=_mesh,
    out_type=jax.ShapeDtypeStruct((NC, NN, DL), jnp.float32),
    scratch_types=[
        pltpu.VMEM((NCH, CHUNK), jnp.int32),
        pltpu.VMEM((2, CHUNK, DL), jnp.float32),
        pltpu.VMEM_SHARED((NN, DL), jnp.float32),
        pltpu.SemaphoreType.DMA,
    ],
)
def _sc_scatter(e_hbm, ridx_hbm, zeros_hbm, out_hbm, ridx_v, rows_v, agg_sh, sr):
    cid = lax.axis_index("c")
    sid = lax.axis_index("s")
    wid = sid * NC + cid
    base = wid * EPT
    pltpu.sync_copy(ridx_hbm.at[wid], ridx_v)

    @pl.when(sid < NS - 1)
    def _():
        pltpu.sync_copy(zeros_hbm.at[pl.ds(0, NPS)],
                        agg_sh.at[pl.ds(sid * NPS, NPS)])

    @pl.when(sid == NS - 1)
    def _():
        pltpu.sync_copy(zeros_hbm, agg_sh.at[pl.ds((NS - 1) * NPS, NPS_LAST)])

    plsc.subcore_barrier()

    def start_read(j, b):
        pltpu.async_copy(e_hbm.at[pl.ds(base + j * CHUNK, CHUNK)],
                         rows_v.at[b], sr)

    def wait_read(j, b):
        pltpu.make_async_copy(e_hbm.at[pl.ds(base + j * CHUNK, CHUNK)],
                              rows_v.at[b], sr).wait()

    start_read(0, 0)

    def body(j, carry):
        b = lax.rem(j, 2)

        @pl.when(j + 1 < NCH)
        def _():
            start_read(j + 1, 1 - b)

        wait_read(j, b)
        pltpu.sync_copy(rows_v.at[b], agg_sh.at[ridx_v.at[j]], add=True)
        return carry

    lax.fori_loop(0, NCH, body, 0)
    plsc.subcore_barrier()

    @pl.when(sid < NS - 1)
    def _():
        pltpu.sync_copy(agg_sh.at[pl.ds(sid * NPS, NPS)],
                        out_hbm.at[cid, pl.ds(sid * NPS, NPS)])

    @pl.when(sid == NS - 1)
    def _():
        pltpu.sync_copy(agg_sh.at[pl.ds((NS - 1) * NPS, NPS_LAST)],
                        out_hbm.at[cid, pl.ds((NS - 1) * NPS, NPS_LAST)])


def kernel(nodes, edges, senders, receivers, params):
    nodes_p = jnp.pad(nodes, ((0, 0), (0, 3)))
    edges_p = jnp.pad(edges, ((0, 0), (0, 3)))
    pne = dict(params['node_enc'])
    pne['W1'] = jnp.pad(params['node_enc']['W1'], ((0, 3), (0, 0)))
    pee = dict(params['edge_enc'])
    pee['W1'] = jnp.pad(params['edge_enc']['W1'], ((0, 3), (0, 0)))

    pe = params['edge_mlp']
    w1e, w1s, w1r, w1ge = (pe['W1'][0:128], pe['W1'][128:256],
                           pe['W1'][256:384], pe['W1'][384:512])
    pn = params['node_mlp']
    w1n, w1a, w1gn = pn['W1'][0:128], pn['W1'][128:256], pn['W1'][256:384]
    pg = params['glob_mlp']
    wgn, wge, wgg = pg['W1'][0:128], pg['W1'][128:256], pg['W1'][256:384]

    sidx = senders.reshape(NW, NCH, CHUNK)
    ridx = receivers.reshape(NW, NCH, CHUNK)
    zeros_slice = jnp.zeros((NPS_LAST, DL), jnp.float32)

    node_lat = _encode(nodes_p, pne, tile=1000)
    edge_lat = _encode(edges_p, pee, tile=2000)
    glob = jnp.zeros((1, DL), jnp.float32)

    for _ in range(4):
        ce, cn = _prep(glob, w1ge, pe['b1'], w1gn, pn['b1'])
        pt, qt = _pq(node_lat, w1s, w1r)
        gp, gq = _sc_gather(pt, qt, sidx, ridx)
        edge_lat, esum = _edge_step(edge_lat, gp, gq, w1e, ce, pe)
        agg2 = _sc_scatter(edge_lat, ridx, zeros_slice)
        node_lat, nsum = _node_step(node_lat, agg2, w1n, w1a, cn, pn)
        glob = _glob_update(nsum, esum, glob, wgn, wge, wgg, pg)

    out = _decode(glob, params['decoder'])
    return out * 1.0 + 0.0


# speedup vs baseline: 1.3218x; 1.0499x over previous
import functools

import jax
import jax.numpy as jnp
from jax import lax
from jax.experimental import pallas as pl
from jax.experimental.pallas import tpu as pltpu
from jax.experimental.pallas import tpu_sc as plsc

NN = 10000
NE = 320000
DL = 128

NC = 2
NS = 16
NW = NC * NS
EPT = NE // NW
CHUNK = 80
NCH = EPT // CHUNK
NPS = 624
NPS_LAST = NN - (NS - 1) * NPS

_mesh = plsc.VectorSubcoreMesh(core_axis_name="c", subcore_axis_name="s")


def _mlp_ln_body(x_ref, w1_ref, b1_ref, w2_ref, b2_ref, sc_ref, of_ref, o_ref):
    h = jnp.dot(x_ref[...], w1_ref[...], preferred_element_type=jnp.float32)
    h = jnp.maximum(h + b1_ref[...], 0.0)
    u = jnp.dot(h, w2_ref[...], preferred_element_type=jnp.float32) + b2_ref[...]
    mu = jnp.mean(u, axis=-1, keepdims=True)
    var = jnp.mean((u - mu) ** 2, axis=-1, keepdims=True)
    o_ref[...] = ((u - mu) * lax.rsqrt(var + 1e-5)) * sc_ref[...] + of_ref[...]


def _encode(x, p, tile):
    n, d = x.shape
    w1 = p['W1']
    full = lambda shape: pl.BlockSpec(shape, lambda i: (0, 0))
    return pl.pallas_call(
        _mlp_ln_body,
        grid=(n // tile,),
        in_specs=[
            pl.BlockSpec((tile, d), lambda i: (i, 0)),
            full((d, DL)), full((1, DL)), full((DL, DL)),
            full((1, DL)), full((1, DL)), full((1, DL)),
        ],
        out_specs=pl.BlockSpec((tile, DL), lambda i: (i, 0)),
        out_shape=jax.ShapeDtypeStruct((n, DL), jnp.float32),
    )(x, w1, p['b1'].reshape(1, -1), p['W2'], p['b2'].reshape(1, -1),
      p['scale'].reshape(1, -1), p['offset'].reshape(1, -1))


def _pq_body(x_ref, ws_ref, wr_ref, p_ref, q_ref):
    x = x_ref[...]
    p_ref[...] = jnp.dot(x, ws_ref[...], preferred_element_type=jnp.float32)
    q_ref[...] = jnp.dot(x, wr_ref[...], preferred_element_type=jnp.float32)


def _pq(node_lat, w1s, w1r, tile=1000):
    full = lambda shape: pl.BlockSpec(shape, lambda i: (0, 0))
    row = pl.BlockSpec((tile, DL), lambda i: (i, 0))
    return pl.pallas_call(
        _pq_body,
        grid=(NN // tile,),
        in_specs=[row, full((DL, DL)), full((DL, DL))],
        out_specs=[row, row],
        out_shape=[jax.ShapeDtypeStruct((NN, DL), jnp.float32)] * 2,
    )(node_lat, w1s, w1r)


def _prep_body(g_ref, wge_ref, b1e_ref, wgn_ref, b1n_ref, ce_ref, cn_ref):
    g = g_ref[...]
    ce_ref[...] = jnp.dot(g, wge_ref[...], preferred_element_type=jnp.float32) + b1e_ref[...]
    cn_ref[...] = jnp.dot(g, wgn_ref[...], preferred_element_type=jnp.float32) + b1n_ref[...]


def _prep(glob, wge, b1e, wgn, b1n):
    full = lambda shape: pl.BlockSpec(shape, lambda: (0, 0))
    return pl.pallas_call(
        _prep_body,
        in_specs=[full((1, DL)), full((DL, DL)), full((1, DL)),
                  full((DL, DL)), full((1, DL))],
        out_specs=[full((1, DL)), full((1, DL))],
        out_shape=[jax.ShapeDtypeStruct((1, DL), jnp.float32)] * 2,
    )(glob, wge, b1e.reshape(1, -1), wgn, b1n.reshape(1, -1))


def _edge_body(e_ref, gp_ref, gq_ref, w1_ref, c_ref, w2_ref, b2_ref,
               sc_ref, of_ref, o_ref, sum_ref):
    e = e_ref[...]
    h = jnp.dot(e, w1_ref[...], preferred_element_type=jnp.float32)
    h = jnp.maximum(h + gp_ref[...] + gq_ref[...] + c_ref[...], 0.0)
    u = jnp.dot(h, w2_ref[...], preferred_element_type=jnp.float32) + b2_ref[...]
    mu = jnp.mean(u, axis=-1, keepdims=True)
    var = jnp.mean((u - mu) ** 2, axis=-1, keepdims=True)
    new = e + ((u - mu) * lax.rsqrt(var + 1e-5)) * sc_ref[...] + of_ref[...]
    o_ref[...] = new

    @pl.when(pl.program_id(0) == 0)
    def _():
        sum_ref[...] = jnp.zeros_like(sum_ref)

    sum_ref[...] += jnp.sum(new, axis=0, keepdims=True)


def _edge_step(edge_lat, gp, gq, w1e, ce, p, tile=2000):
    ne = edge_lat.shape[0]
    full = lambda shape: pl.BlockSpec(shape, lambda i: (0, 0))
    row = pl.BlockSpec((tile, DL), lambda i: (i, 0))
    return pl.pallas_call(
        _edge_body,
        grid=(ne // tile,),
        in_specs=[row, row, row, full((DL, DL)), full((1, DL)),
                  full((DL, DL)), full((1, DL)), full((1, DL)), full((1, DL))],
        out_specs=[row, full((1, DL))],
        out_shape=[jax.ShapeDtypeStruct((ne, DL), jnp.float32),
                   jax.ShapeDtypeStruct((1, DL), jnp.float32)],
    )(edge_lat, gp, gq, w1e, ce, p['W2'], p['b2'].reshape(1, -1),
      p['scale'].reshape(1, -1), p['offset'].reshape(1, -1))


def _node_body(nl_ref, agg_ref, w1n_ref, w1a_ref, c_ref, w2_ref, b2_ref,
               sc_ref, of_ref, o_ref, sum_ref):
    nl = nl_ref[...]
    agg = agg_ref[0] + agg_ref[1]
    h = jnp.dot(nl, w1n_ref[...], preferred_element_type=jnp.float32)
    h = h + jnp.dot(agg, w1a_ref[...], preferred_element_type=jnp.float32)
    h = jnp.maximum(h + c_ref[...], 0.0)
    u = jnp.dot(h, w2_ref[...], preferred_element_type=jnp.float32) + b2_ref[...]
    mu = jnp.mean(u, axis=-1, keepdims=True)
    var = jnp.mean((u - mu) ** 2, axis=-1, keepdims=True)
    new = nl + ((u - mu) * lax.rsqrt(var + 1e-5)) * sc_ref[...] + of_ref[...]
    o_ref[...] = new

    @pl.when(pl.program_id(0) == 0)
    def _():
        sum_ref[...] = jnp.zeros_like(sum_ref)

    sum_ref[...] += jnp.sum(new, axis=0, keepdims=True)


def _node_step(node_lat, agg2, w1n, w1a, cn, p, tile=1000):
    full = lambda shape: pl.BlockSpec(shape, lambda i: (0, 0))
    row = pl.BlockSpec((tile, DL), lambda i: (i, 0))
    return pl.pallas_call(
        _node_body,
        grid=(NN // tile,),
        in_specs=[row, pl.BlockSpec((2, tile, DL), lambda i: (0, i, 0)),
                  full((DL, DL)), full((DL, DL)), full((1, DL)),
                  full((DL, DL)), full((1, DL)), full((1, DL)), full((1, DL))],
        out_specs=[row, full((1, DL))],
        out_shape=[jax.ShapeDtypeStruct((NN, DL), jnp.float32),
                   jax.ShapeDtypeStruct((1, DL), jnp.float32)],
    )(node_lat, agg2, w1n, w1a, cn, p['W2'], p['b2'].reshape(1, -1),
      p['scale'].reshape(1, -1), p['offset'].reshape(1, -1))


def _glob_body(ns_ref, es1_ref, es2_ref, g_ref, wgn_ref, wge_ref, wgg_ref,
               b1_ref, w2_ref, b2_ref, sc_ref, of_ref, o_ref):
    g = g_ref[...]
    es = es1_ref[...] + es2_ref[...]
    h = jnp.dot(ns_ref[...], wgn_ref[...], preferred_element_type=jnp.float32)
    h = h + jnp.dot(es, wge_ref[...], preferred_element_type=jnp.float32)
    h = h + jnp.dot(g, wgg_ref[...], preferred_element_type=jnp.float32)
    h = jnp.maximum(h + b1_ref[...], 0.0)
    u = jnp.dot(h, w2_ref[...], preferred_element_type=jnp.float32) + b2_ref[...]
    mu = jnp.mean(u, axis=-1, keepdims=True)
    var = jnp.mean((u - mu) ** 2, axis=-1, keepdims=True)
    o_ref[...] = g + ((u - mu) * lax.rsqrt(var + 1e-5)) * sc_ref[...] + of_ref[...]


def _glob_update(nsum, esum1, esum2, glob, wgn, wge, wgg, p):
    full = lambda shape: pl.BlockSpec(shape, lambda: (0, 0))
    return pl.pallas_call(
        _glob_body,
        in_specs=[full((1, DL))] * 4 + [full((DL, DL))] * 3 + [full((1, DL)),
                  full((DL, DL)), full((1, DL)), full((1, DL)), full((1, DL))],
        out_specs=full((1, DL)),
        out_shape=jax.ShapeDtypeStruct((1, DL), jnp.float32),
    )(nsum, esum1, esum2, glob, wgn, wge, wgg, p['b1'].reshape(1, -1), p['W2'],
      p['b2'].reshape(1, -1), p['scale'].reshape(1, -1), p['offset'].reshape(1, -1))


def _decode_body(g_ref, w1_ref, b1_ref, w2_ref, b2_ref, o_ref):
    h = jnp.dot(g_ref[...], w1_ref[...], preferred_element_type=jnp.float32)
    h = jnp.maximum(h + b1_ref[...], 0.0)
    o_ref[...] = jnp.dot(h, w2_ref[...], preferred_element_type=jnp.float32) + b2_ref[...]


def _decode(glob, p):
    full = lambda shape: pl.BlockSpec(shape, lambda: (0, 0))
    return pl.pallas_call(
        _decode_body,
        in_specs=[full((1, DL)), full((DL, DL)), full((1, DL)),
                  full((DL, 1)), full((1, 1))],
        out_specs=full((1, 1)),
        out_shape=jax.ShapeDtypeStruct((1, 1), jnp.float32),
    )(glob, p['W1'], p['b1'].reshape(1, -1), p['W2'], p['b2'].reshape(1, -1))


NBUF = 4


def _make_gather(ne, ept, chunk, nch):
    @functools.partial(
        pl.kernel,
        mesh=_mesh,
        out_type=[jax.ShapeDtypeStruct((ne, DL), jnp.float32),
                  jax.ShapeDtypeStruct((ne, DL), jnp.float32)],
        scratch_types=[
            pltpu.VMEM((nch, chunk), jnp.int32),
            pltpu.VMEM((nch, chunk), jnp.int32),
            pltpu.VMEM((NBUF, chunk, DL), jnp.float32),
            pltpu.VMEM((NBUF, chunk, DL), jnp.float32),
            pltpu.SemaphoreType.DMA,
            pltpu.SemaphoreType.DMA,
        ],
    )
    def gather(p_hbm, q_hbm, sidx_hbm, ridx_hbm, gp_hbm, gq_hbm,
               sidx_v, ridx_v, bp, bq, sg, sw):
        wid = lax.axis_index("s") * NC + lax.axis_index("c")
        base = wid * ept
        pltpu.sync_copy(sidx_hbm.at[wid], sidx_v)
        pltpu.sync_copy(ridx_hbm.at[wid], ridx_v)

        def start_gather(j, b):
            pltpu.async_copy(p_hbm.at[sidx_v.at[j]], bp.at[b], sg)
            pltpu.async_copy(q_hbm.at[ridx_v.at[j]], bq.at[b], sg)

        def wait_gather(j, b):
            pltpu.make_async_copy(p_hbm.at[sidx_v.at[j]], bp.at[b], sg).wait()
            pltpu.make_async_copy(q_hbm.at[ridx_v.at[j]], bq.at[b], sg).wait()

        def start_write(j, b):
            sl = pl.ds(base + j * chunk, chunk)
            pltpu.async_copy(bp.at[b], gp_hbm.at[sl], sw)
            pltpu.async_copy(bq.at[b], gq_hbm.at[sl], sw)

        def wait_write(j, b):
            sl = pl.ds(base + j * chunk, chunk)
            pltpu.make_async_copy(bp.at[b], gp_hbm.at[sl], sw).wait()
            pltpu.make_async_copy(bq.at[b], gq_hbm.at[sl], sw).wait()

        LOOK = 2
        for k in range(LOOK):
            start_gather(k, k)

        def body(j, carry):
            b = lax.rem(j, NBUF)

            @pl.when(j >= NBUF - LOOK)
            def _():
                wait_write(j - (NBUF - LOOK), lax.rem(j + LOOK, NBUF))

            @pl.when(j + LOOK < nch)
            def _():
                start_gather(j + LOOK, lax.rem(j + LOOK, NBUF))

            wait_gather(j, b)
            start_write(j, b)
            return carry

        lax.fori_loop(0, nch, body, 0)
        for k in range(NBUF - LOOK):
            j = nch - (NBUF - LOOK) + k
            wait_write(j, j % NBUF)

    return gather


def _make_scatter(ne, ept, chunk, nch):
    @functools.partial(
        pl.kernel,
        mesh=_mesh,
        out_type=jax.ShapeDtypeStruct((NC, NN, DL), jnp.float32),
        scratch_types=[
            pltpu.VMEM((nch, chunk), jnp.int32),
            pltpu.VMEM((2, chunk, DL), jnp.float32),
            pltpu.VMEM_SHARED((NN, DL), jnp.float32),
            pltpu.SemaphoreType.DMA,
        ],
    )
    def scatter(e_hbm, ridx_hbm, init_hbm, out_hbm, ridx_v, rows_v, agg_sh, sr):
        cid = lax.axis_index("c")
        sid = lax.axis_index("s")
        wid = sid * NC + cid
        base = wid * ept
        pltpu.sync_copy(ridx_hbm.at[wid], ridx_v)

        @pl.when(sid < NS - 1)
        def _():
            sl = pl.ds(sid * NPS, NPS)
            pltpu.sync_copy(init_hbm.at[cid, sl], agg_sh.at[sl])

        @pl.when(sid == NS - 1)
        def _():
            sl = pl.ds((NS - 1) * NPS, NPS_LAST)
            pltpu.sync_copy(init_hbm.at[cid, sl], agg_sh.at[sl])

        plsc.subcore_barrier()

        def start_read(j, b):
            pltpu.async_copy(e_hbm.at[pl.ds(base + j * chunk, chunk)],
                             rows_v.at[b], sr)

        def wait_read(j, b):
            pltpu.make_async_copy(e_hbm.at[pl.ds(base + j * chunk, chunk)],
                                  rows_v.at[b], sr).wait()

        start_read(0, 0)

        def body(j, carry):
            b = lax.rem(j, 2)

            @pl.when(j + 1 < nch)
            def _():
                start_read(j + 1, 1 - b)

            wait_read(j, b)
            pltpu.sync_copy(rows_v.at[b], agg_sh.at[ridx_v.at[j]], add=True)
            return carry

        lax.fori_loop(0, nch, body, 0)
        plsc.subcore_barrier()

        @pl.when(sid < NS - 1)
        def _():
            sl = pl.ds(sid * NPS, NPS)
            pltpu.sync_copy(agg_sh.at[sl], out_hbm.at[cid, sl])

        @pl.when(sid == NS - 1)
        def _():
            sl = pl.ds((NS - 1) * NPS, NPS_LAST)
            pltpu.sync_copy(agg_sh.at[sl], out_hbm.at[cid, sl])

    return scatter


NE2 = NE // 2
EPT_H = NE2 // NW
CHUNK_H = 40
NCH_H = EPT_H // CHUNK_H

_gather_half = _make_gather(NE2, EPT_H, CHUNK_H, NCH_H)
_scatter_half = _make_scatter(NE2, EPT_H, CHUNK_H, NCH_H)


def kernel(nodes, edges, senders, receivers, params):
    nodes_p = jnp.pad(nodes, ((0, 0), (0, 3)))
    edges_p = jnp.pad(edges, ((0, 0), (0, 3)))
    pne = dict(params['node_enc'])
    pne['W1'] = jnp.pad(params['node_enc']['W1'], ((0, 3), (0, 0)))
    pee = dict(params['edge_enc'])
    pee['W1'] = jnp.pad(params['edge_enc']['W1'], ((0, 3), (0, 0)))

    pe = params['edge_mlp']
    w1e, w1s, w1r, w1ge = (pe['W1'][0:128], pe['W1'][128:256],
                           pe['W1'][256:384], pe['W1'][384:512])
    pn = params['node_mlp']
    w1n, w1a, w1gn = pn['W1'][0:128], pn['W1'][128:256], pn['W1'][256:384]
    pg = params['glob_mlp']
    wgn, wge, wgg = pg['W1'][0:128], pg['W1'][128:256], pg['W1'][256:384]

    sidx = [senders[:NE2].reshape(NW, NCH_H, CHUNK_H),
            senders[NE2:].reshape(NW, NCH_H, CHUNK_H)]
    ridx = [receivers[:NE2].reshape(NW, NCH_H, CHUNK_H),
            receivers[NE2:].reshape(NW, NCH_H, CHUNK_H)]
    agg_zero = jnp.zeros((NC, NN, DL), jnp.float32)

    node_lat = _encode(nodes_p, pne, tile=1000)
    elat = [_encode(edges_p[:NE2], pee, tile=2000),
            _encode(edges_p[NE2:], pee, tile=2000)]
    glob = jnp.zeros((1, DL), jnp.float32)

    for _ in range(4):
        ce, cn = _prep(glob, w1ge, pe['b1'], w1gn, pn['b1'])
        pt, qt = _pq(node_lat, w1s, w1r)
        gpA, gqA = _gather_half(pt, qt, sidx[0], ridx[0])
        gpB, gqB = _gather_half(pt, qt, sidx[1], ridx[1])
        elat[0], esA = _edge_step(elat[0], gpA, gqA, w1e, ce, pe)
        aggA = _scatter_half(elat[0], ridx[0], agg_zero)
        elat[1], esB = _edge_step(elat[1], gpB, gqB, w1e, ce, pe)
        agg2 = _scatter_half(elat[1], ridx[1], aggA)
        node_lat, nsum = _node_step(node_lat, agg2, w1n, w1a, cn, pn)
        glob = _glob_update(nsum, esA, esB, glob, wgn, wge, wgg, pg)

    out = _decode(glob, params['decoder'])
    return out * 1.0 + 0.0


# speedup vs baseline: 1.3836x; 1.0468x over previous
import functools

import jax
import jax.numpy as jnp
from jax import lax
from jax.experimental import pallas as pl
from jax.experimental.pallas import tpu as pltpu
from jax.experimental.pallas import tpu_sc as plsc

NN = 10000
NE = 320000
DL = 128

NC = 2
NS = 16
NW = NC * NS
EPT = NE // NW
CHUNK = 80
NCH = EPT // CHUNK
NPS = 624
NPS_LAST = NN - (NS - 1) * NPS

_mesh = plsc.VectorSubcoreMesh(core_axis_name="c", subcore_axis_name="s")


def _mlp_ln_body(x_ref, w1_ref, b1_ref, w2_ref, b2_ref, sc_ref, of_ref, o_ref):
    h = jnp.dot(x_ref[...], w1_ref[...], preferred_element_type=jnp.float32)
    h = jnp.maximum(h + b1_ref[...], 0.0)
    u = jnp.dot(h, w2_ref[...], preferred_element_type=jnp.float32) + b2_ref[...]
    mu = jnp.mean(u, axis=-1, keepdims=True)
    var = jnp.mean((u - mu) ** 2, axis=-1, keepdims=True)
    o_ref[...] = ((u - mu) * lax.rsqrt(var + 1e-5)) * sc_ref[...] + of_ref[...]


def _encode(x, p, tile):
    n, d = x.shape
    w1 = p['W1']
    full = lambda shape: pl.BlockSpec(shape, lambda i: (0, 0))
    return pl.pallas_call(
        _mlp_ln_body,
        grid=(n // tile,),
        in_specs=[
            pl.BlockSpec((tile, d), lambda i: (i, 0)),
            full((d, DL)), full((1, DL)), full((DL, DL)),
            full((1, DL)), full((1, DL)), full((1, DL)),
        ],
        out_specs=pl.BlockSpec((tile, DL), lambda i: (i, 0)),
        out_shape=jax.ShapeDtypeStruct((n, DL), jnp.float32),
    )(x, w1, p['b1'].reshape(1, -1), p['W2'], p['b2'].reshape(1, -1),
      p['scale'].reshape(1, -1), p['offset'].reshape(1, -1))


def _pq_body(x_ref, ws_ref, wr_ref, p_ref, q_ref):
    x = x_ref[...]
    p_ref[...] = jnp.dot(x, ws_ref[...], preferred_element_type=jnp.float32)
    q_ref[...] = jnp.dot(x, wr_ref[...], preferred_element_type=jnp.float32)


def _pq(node_lat, w1s, w1r, tile=1000):
    full = lambda shape: pl.BlockSpec(shape, lambda i: (0, 0))
    row = pl.BlockSpec((tile, DL), lambda i: (i, 0))
    return pl.pallas_call(
        _pq_body,
        grid=(NN // tile,),
        in_specs=[row, full((DL, DL)), full((DL, DL))],
        out_specs=[row, row],
        out_shape=[jax.ShapeDtypeStruct((NN, DL), jnp.float32)] * 2,
    )(node_lat, w1s, w1r)


def _prep_body(g_ref, wge_ref, b1e_ref, wgn_ref, b1n_ref, ce_ref, cn_ref):
    g = g_ref[...]
    ce_ref[...] = jnp.dot(g, wge_ref[...], preferred_element_type=jnp.float32) + b1e_ref[...]
    cn_ref[...] = jnp.dot(g, wgn_ref[...], preferred_element_type=jnp.float32) + b1n_ref[...]


def _prep(glob, wge, b1e, wgn, b1n):
    full = lambda shape: pl.BlockSpec(shape, lambda: (0, 0))
    return pl.pallas_call(
        _prep_body,
        in_specs=[full((1, DL)), full((DL, DL)), full((1, DL)),
                  full((DL, DL)), full((1, DL))],
        out_specs=[full((1, DL)), full((1, DL))],
        out_shape=[jax.ShapeDtypeStruct((1, DL), jnp.float32)] * 2,
    )(glob, wge, b1e.reshape(1, -1), wgn, b1n.reshape(1, -1))


def _edge_body(e_ref, gp_ref, gq_ref, w1_ref, c_ref, w2_ref, b2_ref,
               sc_ref, of_ref, o_ref, sum_ref):
    e = e_ref[...]
    h = jnp.dot(e, w1_ref[...], preferred_element_type=jnp.float32)
    h = jnp.maximum(h + gp_ref[...] + gq_ref[...] + c_ref[...], 0.0)
    u = jnp.dot(h, w2_ref[...], preferred_element_type=jnp.float32) + b2_ref[...]
    mu = jnp.mean(u, axis=-1, keepdims=True)
    var = jnp.mean((u - mu) ** 2, axis=-1, keepdims=True)
    new = e + ((u - mu) * lax.rsqrt(var + 1e-5)) * sc_ref[...] + of_ref[...]
    o_ref[...] = new

    @pl.when(pl.program_id(0) == 0)
    def _():
        sum_ref[...] = jnp.zeros_like(sum_ref)

    sum_ref[...] += jnp.sum(new, axis=0, keepdims=True)


def _edge_step(edge_lat, gp, gq, w1e, ce, p, tile=2000):
    ne = edge_lat.shape[0]
    full = lambda shape: pl.BlockSpec(shape, lambda i: (0, 0))
    row = pl.BlockSpec((tile, DL), lambda i: (i, 0))
    return pl.pallas_call(
        _edge_body,
        grid=(ne // tile,),
        in_specs=[row, row, row, full((DL, DL)), full((1, DL)),
                  full((DL, DL)), full((1, DL)), full((1, DL)), full((1, DL))],
        out_specs=[row, full((1, DL))],
        out_shape=[jax.ShapeDtypeStruct((ne, DL), jnp.float32),
                   jax.ShapeDtypeStruct((1, DL), jnp.float32)],
    )(edge_lat, gp, gq, w1e, ce, p['W2'], p['b2'].reshape(1, -1),
      p['scale'].reshape(1, -1), p['offset'].reshape(1, -1))


def _node_body(nl_ref, agg_ref, w1n_ref, w1a_ref, c_ref, w2_ref, b2_ref,
               sc_ref, of_ref, o_ref, sum_ref):
    nl = nl_ref[...]
    agg = agg_ref[0] + agg_ref[1]
    h = jnp.dot(nl, w1n_ref[...], preferred_element_type=jnp.float32)
    h = h + jnp.dot(agg, w1a_ref[...], preferred_element_type=jnp.float32)
    h = jnp.maximum(h + c_ref[...], 0.0)
    u = jnp.dot(h, w2_ref[...], preferred_element_type=jnp.float32) + b2_ref[...]
    mu = jnp.mean(u, axis=-1, keepdims=True)
    var = jnp.mean((u - mu) ** 2, axis=-1, keepdims=True)
    new = nl + ((u - mu) * lax.rsqrt(var + 1e-5)) * sc_ref[...] + of_ref[...]
    o_ref[...] = new

    @pl.when(pl.program_id(0) == 0)
    def _():
        sum_ref[...] = jnp.zeros_like(sum_ref)

    sum_ref[...] += jnp.sum(new, axis=0, keepdims=True)


def _node_step(node_lat, agg2, w1n, w1a, cn, p, tile=1000):
    full = lambda shape: pl.BlockSpec(shape, lambda i: (0, 0))
    row = pl.BlockSpec((tile, DL), lambda i: (i, 0))
    return pl.pallas_call(
        _node_body,
        grid=(NN // tile,),
        in_specs=[row, pl.BlockSpec((2, tile, DL), lambda i: (0, i, 0)),
                  full((DL, DL)), full((DL, DL)), full((1, DL)),
                  full((DL, DL)), full((1, DL)), full((1, DL)), full((1, DL))],
        out_specs=[row, full((1, DL))],
        out_shape=[jax.ShapeDtypeStruct((NN, DL), jnp.float32),
                   jax.ShapeDtypeStruct((1, DL), jnp.float32)],
    )(node_lat, agg2, w1n, w1a, cn, p['W2'], p['b2'].reshape(1, -1),
      p['scale'].reshape(1, -1), p['offset'].reshape(1, -1))


def _glob_body(ns_ref, es1_ref, es2_ref, g_ref, wgn_ref, wge_ref, wgg_ref,
               b1_ref, w2_ref, b2_ref, sc_ref, of_ref, o_ref):
    g = g_ref[...]
    es = es1_ref[...] + es2_ref[...]
    h = jnp.dot(ns_ref[...], wgn_ref[...], preferred_element_type=jnp.float32)
    h = h + jnp.dot(es, wge_ref[...], preferred_element_type=jnp.float32)
    h = h + jnp.dot(g, wgg_ref[...], preferred_element_type=jnp.float32)
    h = jnp.maximum(h + b1_ref[...], 0.0)
    u = jnp.dot(h, w2_ref[...], preferred_element_type=jnp.float32) + b2_ref[...]
    mu = jnp.mean(u, axis=-1, keepdims=True)
    var = jnp.mean((u - mu) ** 2, axis=-1, keepdims=True)
    o_ref[...] = g + ((u - mu) * lax.rsqrt(var + 1e-5)) * sc_ref[...] + of_ref[...]


def _glob_update(nsum, esum1, esum2, glob, wgn, wge, wgg, p):
    full = lambda shape: pl.BlockSpec(shape, lambda: (0, 0))
    return pl.pallas_call(
        _glob_body,
        in_specs=[full((1, DL))] * 4 + [full((DL, DL))] * 3 + [full((1, DL)),
                  full((DL, DL)), full((1, DL)), full((1, DL)), full((1, DL))],
        out_specs=full((1, DL)),
        out_shape=jax.ShapeDtypeStruct((1, DL), jnp.float32),
    )(nsum, esum1, esum2, glob, wgn, wge, wgg, p['b1'].reshape(1, -1), p['W2'],
      p['b2'].reshape(1, -1), p['scale'].reshape(1, -1), p['offset'].reshape(1, -1))


def _decode_body(g_ref, w1_ref, b1_ref, w2_ref, b2_ref, o_ref):
    h = jnp.dot(g_ref[...], w1_ref[...], preferred_element_type=jnp.float32)
    h = jnp.maximum(h + b1_ref[...], 0.0)
    o_ref[...] = jnp.dot(h, w2_ref[...], preferred_element_type=jnp.float32) + b2_ref[...]


def _decode(glob, p):
    full = lambda shape: pl.BlockSpec(shape, lambda: (0, 0))
    return pl.pallas_call(
        _decode_body,
        in_specs=[full((1, DL)), full((DL, DL)), full((1, DL)),
                  full((DL, 1)), full((1, 1))],
        out_specs=full((1, 1)),
        out_shape=jax.ShapeDtypeStruct((1, 1), jnp.float32),
    )(glob, p['W1'], p['b1'].reshape(1, -1), p['W2'], p['b2'].reshape(1, -1))


NBUF = 4


def _make_gather(ne, ept, chunk, nch):
    @functools.partial(
        pl.kernel,
        mesh=_mesh,
        out_type=[jax.ShapeDtypeStruct((ne, DL), jnp.float32),
                  jax.ShapeDtypeStruct((ne, DL), jnp.float32)],
        scratch_types=[
            pltpu.VMEM((nch, chunk), jnp.int32),
            pltpu.VMEM((nch, chunk), jnp.int32),
            pltpu.VMEM((NBUF, chunk, DL), jnp.float32),
            pltpu.VMEM((NBUF, chunk, DL), jnp.float32),
            pltpu.SemaphoreType.DMA,
            pltpu.SemaphoreType.DMA,
        ],
    )
    def gather(p_hbm, q_hbm, sidx_hbm, ridx_hbm, gp_hbm, gq_hbm,
               sidx_v, ridx_v, bp, bq, sg, sw):
        wid = lax.axis_index("s") * NC + lax.axis_index("c")
        base = wid * ept
        pltpu.sync_copy(sidx_hbm.at[wid], sidx_v)
        pltpu.sync_copy(ridx_hbm.at[wid], ridx_v)

        def start_gather(j, b):
            pltpu.async_copy(p_hbm.at[sidx_v.at[j]], bp.at[b], sg)
            pltpu.async_copy(q_hbm.at[ridx_v.at[j]], bq.at[b], sg)

        def wait_gather(j, b):
            pltpu.make_async_copy(p_hbm.at[sidx_v.at[j]], bp.at[b], sg).wait()
            pltpu.make_async_copy(q_hbm.at[ridx_v.at[j]], bq.at[b], sg).wait()

        def start_write(j, b):
            sl = pl.ds(base + j * chunk, chunk)
            pltpu.async_copy(bp.at[b], gp_hbm.at[sl], sw)
            pltpu.async_copy(bq.at[b], gq_hbm.at[sl], sw)

        def wait_write(j, b):
            sl = pl.ds(base + j * chunk, chunk)
            pltpu.make_async_copy(bp.at[b], gp_hbm.at[sl], sw).wait()
            pltpu.make_async_copy(bq.at[b], gq_hbm.at[sl], sw).wait()

        LOOK = 2
        for k in range(LOOK):
            start_gather(k, k)

        def body(j, carry):
            b = lax.rem(j, NBUF)

            @pl.when(j >= NBUF - LOOK)
            def _():
                wait_write(j - (NBUF - LOOK), lax.rem(j + LOOK, NBUF))

            @pl.when(j + LOOK < nch)
            def _():
                start_gather(j + LOOK, lax.rem(j + LOOK, NBUF))

            wait_gather(j, b)
            start_write(j, b)
            return carry

        lax.fori_loop(0, nch, body, 0)
        for k in range(NBUF - LOOK):
            j = nch - (NBUF - LOOK) + k
            wait_write(j, j % NBUF)

    return gather


def _make_scatter(ne, ept, chunk, nch):
    @functools.partial(
        pl.kernel,
        mesh=_mesh,
        out_type=jax.ShapeDtypeStruct((NC, NN, DL), jnp.float32),
        scratch_types=[
            pltpu.VMEM((nch, chunk), jnp.int32),
            pltpu.VMEM((2, chunk, DL), jnp.float32),
            pltpu.VMEM_SHARED((NN, DL), jnp.float32),
            pltpu.SemaphoreType.DMA,
        ],
    )
    def scatter(e_hbm, ridx_hbm, init_hbm, out_hbm, ridx_v, rows_v, agg_sh, sr):
        cid = lax.axis_index("c")
        sid = lax.axis_index("s")
        wid = sid * NC + cid
        base = wid * ept
        pltpu.sync_copy(ridx_hbm.at[wid], ridx_v)

        @pl.when(sid < NS - 1)
        def _():
            sl = pl.ds(sid * NPS, NPS)
            pltpu.sync_copy(init_hbm.at[cid, sl], agg_sh.at[sl])

        @pl.when(sid == NS - 1)
        def _():
            sl = pl.ds((NS - 1) * NPS, NPS_LAST)
            pltpu.sync_copy(init_hbm.at[cid, sl], agg_sh.at[sl])

        plsc.subcore_barrier()

        def start_read(j, b):
            pltpu.async_copy(e_hbm.at[pl.ds(base + j * chunk, chunk)],
                             rows_v.at[b], sr)

        def wait_read(j, b):
            pltpu.make_async_copy(e_hbm.at[pl.ds(base + j * chunk, chunk)],
                                  rows_v.at[b], sr).wait()

        start_read(0, 0)

        def body(j, carry):
            b = lax.rem(j, 2)

            @pl.when(j + 1 < nch)
            def _():
                start_read(j + 1, 1 - b)

            wait_read(j, b)
            pltpu.sync_copy(rows_v.at[b], agg_sh.at[ridx_v.at[j]], add=True)
            return carry

        lax.fori_loop(0, nch, body, 0)
        plsc.subcore_barrier()

        @pl.when(sid < NS - 1)
        def _():
            sl = pl.ds(sid * NPS, NPS)
            pltpu.sync_copy(agg_sh.at[sl], out_hbm.at[cid, sl])

        @pl.when(sid == NS - 1)
        def _():
            sl = pl.ds((NS - 1) * NPS, NPS_LAST)
            pltpu.sync_copy(agg_sh.at[sl], out_hbm.at[cid, sl])

    return scatter


NE_A = 62 * CHUNK * NW
NE_B = NE - NE_A
EPT_A, NCH_A = NE_A // NW, 62
EPT_B, NCH_B = NE_B // NW, 63
TILE_A = NE_A // 64
TILE_B = NE_B // 64

_gather_a = _make_gather(NE_A, EPT_A, CHUNK, NCH_A)
_gather_b = _make_gather(NE_B, EPT_B, CHUNK, NCH_B)
_scatter_a = _make_scatter(NE_A, EPT_A, CHUNK, NCH_A)
_scatter_b = _make_scatter(NE_B, EPT_B, CHUNK, NCH_B)


def kernel(nodes, edges, senders, receivers, params):
    nodes_p = jnp.pad(nodes, ((0, 0), (0, 3)))
    edges_p = jnp.pad(edges, ((0, 0), (0, 3)))
    pne = dict(params['node_enc'])
    pne['W1'] = jnp.pad(params['node_enc']['W1'], ((0, 3), (0, 0)))
    pee = dict(params['edge_enc'])
    pee['W1'] = jnp.pad(params['edge_enc']['W1'], ((0, 3), (0, 0)))

    pe = params['edge_mlp']
    w1e, w1s, w1r, w1ge = (pe['W1'][0:128], pe['W1'][128:256],
                           pe['W1'][256:384], pe['W1'][384:512])
    pn = params['node_mlp']
    w1n, w1a, w1gn = pn['W1'][0:128], pn['W1'][128:256], pn['W1'][256:384]
    pg = params['glob_mlp']
    wgn, wge, wgg = pg['W1'][0:128], pg['W1'][128:256], pg['W1'][256:384]

    sidx = [senders[:NE_A].reshape(NW, NCH_A, CHUNK),
            senders[NE_A:].reshape(NW, NCH_B, CHUNK)]
    ridx = [receivers[:NE_A].reshape(NW, NCH_A, CHUNK),
            receivers[NE_A:].reshape(NW, NCH_B, CHUNK)]
    agg_zero = jnp.zeros((NC, NN, DL), jnp.float32)

    node_lat = _encode(nodes_p, pne, tile=1000)
    elat = [_encode(edges_p[:NE_A], pee, tile=TILE_A),
            _encode(edges_p[NE_A:], pee, tile=TILE_B)]
    glob = jnp.zeros((1, DL), jnp.float32)

    for _ in range(4):
        ce, cn = _prep(glob, w1ge, pe['b1'], w1gn, pn['b1'])
        pt, qt = _pq(node_lat, w1s, w1r)
        gpA, gqA = _gather_a(pt, qt, sidx[0], ridx[0])
        gpB, gqB = _gather_b(pt, qt, sidx[1], ridx[1])
        elat[0], esA = _edge_step(elat[0], gpA, gqA, w1e, ce, pe, tile=TILE_A)
        aggA = _scatter_a(elat[0], ridx[0], agg_zero)
        elat[1], esB = _edge_step(elat[1], gpB, gqB, w1e, ce, pe, tile=TILE_B)
        agg2 = _scatter_b(elat[1], ridx[1], aggA)
        node_lat, nsum = _node_step(node_lat, agg2, w1n, w1a, cn, pn)
        glob = _glob_update(nsum, esA, esB, glob, wgn, wge, wgg, pg)

    out = _decode(glob, params['decoder'])
    return out * 1.0 + 0.0


# speedup vs baseline: 1.4409x; 1.0414x over previous
import functools

import jax
import jax.numpy as jnp
from jax import lax
from jax.experimental import pallas as pl
from jax.experimental.pallas import tpu as pltpu
from jax.experimental.pallas import tpu_sc as plsc

NN = 10000
NE = 320000
DL = 128

NC = 2
NS = 16
NW = NC * NS
EPT = NE // NW
CHUNK = 80
NCH = EPT // CHUNK
NPS = 624
NPS_LAST = NN - (NS - 1) * NPS

_mesh = plsc.VectorSubcoreMesh(core_axis_name="c", subcore_axis_name="s")


def _mlp_ln_body(x_ref, w1_ref, b1_ref, w2_ref, b2_ref, sc_ref, of_ref, o_ref):
    h = jnp.dot(x_ref[...], w1_ref[...], preferred_element_type=jnp.float32)
    h = jnp.maximum(h + b1_ref[...], 0.0)
    u = jnp.dot(h, w2_ref[...], preferred_element_type=jnp.float32) + b2_ref[...]
    mu = jnp.mean(u, axis=-1, keepdims=True)
    var = jnp.mean((u - mu) ** 2, axis=-1, keepdims=True)
    o_ref[...] = ((u - mu) * lax.rsqrt(var + 1e-5)) * sc_ref[...] + of_ref[...]


def _encode(x, p, tile):
    n, d = x.shape
    w1 = p['W1']
    full = lambda shape: pl.BlockSpec(shape, lambda i: (0, 0))
    return pl.pallas_call(
        _mlp_ln_body,
        grid=(n // tile,),
        in_specs=[
            pl.BlockSpec((tile, d), lambda i: (i, 0)),
            full((d, DL)), full((1, DL)), full((DL, DL)),
            full((1, DL)), full((1, DL)), full((1, DL)),
        ],
        out_specs=pl.BlockSpec((tile, DL), lambda i: (i, 0)),
        out_shape=jax.ShapeDtypeStruct((n, DL), jnp.float32),
    )(x, w1, p['b1'].reshape(1, -1), p['W2'], p['b2'].reshape(1, -1),
      p['scale'].reshape(1, -1), p['offset'].reshape(1, -1))


def _pq_body(x_ref, ws_ref, wr_ref, p_ref, q_ref):
    x = x_ref[...]
    p_ref[...] = jnp.dot(x, ws_ref[...], preferred_element_type=jnp.float32)
    q_ref[...] = jnp.dot(x, wr_ref[...], preferred_element_type=jnp.float32)


def _pq(node_lat, w1s, w1r, tile=1000):
    full = lambda shape: pl.BlockSpec(shape, lambda i: (0, 0))
    row = pl.BlockSpec((tile, DL), lambda i: (i, 0))
    return pl.pallas_call(
        _pq_body,
        grid=(NN // tile,),
        in_specs=[row, full((DL, DL)), full((DL, DL))],
        out_specs=[row, row],
        out_shape=[jax.ShapeDtypeStruct((NN, DL), jnp.float32)] * 2,
    )(node_lat, w1s, w1r)


def _prep_body(g_ref, wge_ref, b1e_ref, wgn_ref, b1n_ref, ce_ref, cn_ref):
    g = g_ref[...]
    ce_ref[...] = jnp.dot(g, wge_ref[...], preferred_element_type=jnp.float32) + b1e_ref[...]
    cn_ref[...] = jnp.dot(g, wgn_ref[...], preferred_element_type=jnp.float32) + b1n_ref[...]


def _prep(glob, wge, b1e, wgn, b1n):
    full = lambda shape: pl.BlockSpec(shape, lambda: (0, 0))
    return pl.pallas_call(
        _prep_body,
        in_specs=[full((1, DL)), full((DL, DL)), full((1, DL)),
                  full((DL, DL)), full((1, DL))],
        out_specs=[full((1, DL)), full((1, DL))],
        out_shape=[jax.ShapeDtypeStruct((1, DL), jnp.float32)] * 2,
    )(glob, wge, b1e.reshape(1, -1), wgn, b1n.reshape(1, -1))


def _edge_body(e_ref, gp_ref, gq_ref, w1_ref, c_ref, w2_ref, b2_ref,
               sc_ref, of_ref, o_ref, sum_ref):
    e = e_ref[...]
    h = jnp.dot(e, w1_ref[...], preferred_element_type=jnp.float32)
    h = jnp.maximum(h + gp_ref[...] + gq_ref[...] + c_ref[...], 0.0)
    u = jnp.dot(h, w2_ref[...], preferred_element_type=jnp.float32) + b2_ref[...]
    mu = jnp.mean(u, axis=-1, keepdims=True)
    var = jnp.mean((u - mu) ** 2, axis=-1, keepdims=True)
    new = e + ((u - mu) * lax.rsqrt(var + 1e-5)) * sc_ref[...] + of_ref[...]
    o_ref[...] = new

    @pl.when(pl.program_id(0) == 0)
    def _():
        sum_ref[...] = jnp.zeros_like(sum_ref)

    sum_ref[...] += jnp.sum(new, axis=0, keepdims=True)


def _edge_step(edge_lat, gp, gq, w1e, ce, p, tile=2000):
    ne = edge_lat.shape[0]
    full = lambda shape: pl.BlockSpec(shape, lambda i: (0, 0))
    row = pl.BlockSpec((tile, DL), lambda i: (i, 0))
    return pl.pallas_call(
        _edge_body,
        grid=(ne // tile,),
        in_specs=[row, row, row, full((DL, DL)), full((1, DL)),
                  full((DL, DL)), full((1, DL)), full((1, DL)), full((1, DL))],
        out_specs=[row, full((1, DL))],
        out_shape=[jax.ShapeDtypeStruct((ne, DL), jnp.float32),
                   jax.ShapeDtypeStruct((1, DL), jnp.float32)],
    )(edge_lat, gp, gq, w1e, ce, p['W2'], p['b2'].reshape(1, -1),
      p['scale'].reshape(1, -1), p['offset'].reshape(1, -1))


def _node_body(nl_ref, agg_ref, w1n_ref, w1a_ref, c_ref, w2_ref, b2_ref,
               sc_ref, of_ref, o_ref, sum_ref):
    nl = nl_ref[...]
    agg = agg_ref[0] + agg_ref[1]
    h = jnp.dot(nl, w1n_ref[...], preferred_element_type=jnp.float32)
    h = h + jnp.dot(agg, w1a_ref[...], preferred_element_type=jnp.float32)
    h = jnp.maximum(h + c_ref[...], 0.0)
    u = jnp.dot(h, w2_ref[...], preferred_element_type=jnp.float32) + b2_ref[...]
    mu = jnp.mean(u, axis=-1, keepdims=True)
    var = jnp.mean((u - mu) ** 2, axis=-1, keepdims=True)
    new = nl + ((u - mu) * lax.rsqrt(var + 1e-5)) * sc_ref[...] + of_ref[...]
    o_ref[...] = new

    @pl.when(pl.program_id(0) == 0)
    def _():
        sum_ref[...] = jnp.zeros_like(sum_ref)

    sum_ref[...] += jnp.sum(new, axis=0, keepdims=True)


def _node_step(node_lat, agg2, w1n, w1a, cn, p, tile=1000):
    full = lambda shape: pl.BlockSpec(shape, lambda i: (0, 0))
    row = pl.BlockSpec((tile, DL), lambda i: (i, 0))
    return pl.pallas_call(
        _node_body,
        grid=(NN // tile,),
        in_specs=[row, pl.BlockSpec((2, tile, DL), lambda i: (0, i, 0)),
                  full((DL, DL)), full((DL, DL)), full((1, DL)),
                  full((DL, DL)), full((1, DL)), full((1, DL)), full((1, DL))],
        out_specs=[row, full((1, DL))],
        out_shape=[jax.ShapeDtypeStruct((NN, DL), jnp.float32),
                   jax.ShapeDtypeStruct((1, DL), jnp.float32)],
    )(node_lat, agg2, w1n, w1a, cn, p['W2'], p['b2'].reshape(1, -1),
      p['scale'].reshape(1, -1), p['offset'].reshape(1, -1))


def _glob_body(ns_ref, es1_ref, es2_ref, g_ref, wgn_ref, wge_ref, wgg_ref,
               b1_ref, w2_ref, b2_ref, sc_ref, of_ref, o_ref):
    g = g_ref[...]
    es = es1_ref[...] + es2_ref[...]
    h = jnp.dot(ns_ref[...], wgn_ref[...], preferred_element_type=jnp.float32)
    h = h + jnp.dot(es, wge_ref[...], preferred_element_type=jnp.float32)
    h = h + jnp.dot(g, wgg_ref[...], preferred_element_type=jnp.float32)
    h = jnp.maximum(h + b1_ref[...], 0.0)
    u = jnp.dot(h, w2_ref[...], preferred_element_type=jnp.float32) + b2_ref[...]
    mu = jnp.mean(u, axis=-1, keepdims=True)
    var = jnp.mean((u - mu) ** 2, axis=-1, keepdims=True)
    o_ref[...] = g + ((u - mu) * lax.rsqrt(var + 1e-5)) * sc_ref[...] + of_ref[...]


def _glob_update(nsum, esum1, esum2, glob, wgn, wge, wgg, p):
    full = lambda shape: pl.BlockSpec(shape, lambda: (0, 0))
    return pl.pallas_call(
        _glob_body,
        in_specs=[full((1, DL))] * 4 + [full((DL, DL))] * 3 + [full((1, DL)),
                  full((DL, DL)), full((1, DL)), full((1, DL)), full((1, DL))],
        out_specs=full((1, DL)),
        out_shape=jax.ShapeDtypeStruct((1, DL), jnp.float32),
    )(nsum, esum1, esum2, glob, wgn, wge, wgg, p['b1'].reshape(1, -1), p['W2'],
      p['b2'].reshape(1, -1), p['scale'].reshape(1, -1), p['offset'].reshape(1, -1))


def _decode_body(g_ref, w1_ref, b1_ref, w2_ref, b2_ref, o_ref):
    h = jnp.dot(g_ref[...], w1_ref[...], preferred_element_type=jnp.float32)
    h = jnp.maximum(h + b1_ref[...], 0.0)
    o_ref[...] = jnp.dot(h, w2_ref[...], preferred_element_type=jnp.float32) + b2_ref[...]


def _decode(glob, p):
    full = lambda shape: pl.BlockSpec(shape, lambda: (0, 0))
    return pl.pallas_call(
        _decode_body,
        in_specs=[full((1, DL)), full((DL, DL)), full((1, DL)),
                  full((DL, 1)), full((1, 1))],
        out_specs=full((1, 1)),
        out_shape=jax.ShapeDtypeStruct((1, 1), jnp.float32),
    )(glob, p['W1'], p['b1'].reshape(1, -1), p['W2'], p['b2'].reshape(1, -1))


NBUF = 4


def _make_gather(ne, ept, chunk, nch):
    @functools.partial(
        pl.kernel,
        mesh=_mesh,
        out_type=[jax.ShapeDtypeStruct((ne, DL), jnp.float32),
                  jax.ShapeDtypeStruct((ne, DL), jnp.float32)],
        scratch_types=[
            pltpu.VMEM((nch, chunk), jnp.int32),
            pltpu.VMEM((nch, chunk), jnp.int32),
            pltpu.VMEM((NBUF, chunk, DL), jnp.float32),
            pltpu.VMEM((NBUF, chunk, DL), jnp.float32),
            pltpu.SemaphoreType.DMA,
            pltpu.SemaphoreType.DMA,
        ],
    )
    def gather(p_hbm, q_hbm, sidx_hbm, ridx_hbm, gp_hbm, gq_hbm,
               sidx_v, ridx_v, bp, bq, sg, sw):
        wid = lax.axis_index("s") * NC + lax.axis_index("c")
        base = wid * ept
        pltpu.sync_copy(sidx_hbm.at[wid], sidx_v)
        pltpu.sync_copy(ridx_hbm.at[wid], ridx_v)

        def start_gather(j, b):
            pltpu.async_copy(p_hbm.at[sidx_v.at[j]], bp.at[b], sg)
            pltpu.async_copy(q_hbm.at[ridx_v.at[j]], bq.at[b], sg)

        def wait_gather(j, b):
            pltpu.make_async_copy(p_hbm.at[sidx_v.at[j]], bp.at[b], sg).wait()
            pltpu.make_async_copy(q_hbm.at[ridx_v.at[j]], bq.at[b], sg).wait()

        def start_write(j, b):
            sl = pl.ds(base + j * chunk, chunk)
            pltpu.async_copy(bp.at[b], gp_hbm.at[sl], sw)
            pltpu.async_copy(bq.at[b], gq_hbm.at[sl], sw)

        def wait_write(j, b):
            sl = pl.ds(base + j * chunk, chunk)
            pltpu.make_async_copy(bp.at[b], gp_hbm.at[sl], sw).wait()
            pltpu.make_async_copy(bq.at[b], gq_hbm.at[sl], sw).wait()

        LOOK = 2
        for k in range(LOOK):
            start_gather(k, k)

        def body(j, carry):
            b = lax.rem(j, NBUF)

            @pl.when(j >= NBUF - LOOK)
            def _():
                wait_write(j - (NBUF - LOOK), lax.rem(j + LOOK, NBUF))

            @pl.when(j + LOOK < nch)
            def _():
                start_gather(j + LOOK, lax.rem(j + LOOK, NBUF))

            wait_gather(j, b)
            start_write(j, b)
            return carry

        lax.fori_loop(0, nch, body, 0)
        for k in range(NBUF - LOOK):
            j = nch - (NBUF - LOOK) + k
            wait_write(j, j % NBUF)

    return gather


def _make_scatter(ne, ept, chunk, nch):
    @functools.partial(
        pl.kernel,
        mesh=_mesh,
        out_type=jax.ShapeDtypeStruct((NC, NN, DL), jnp.float32),
        scratch_types=[
            pltpu.VMEM((nch, chunk), jnp.int32),
            pltpu.VMEM((2, chunk, DL), jnp.float32),
            pltpu.VMEM_SHARED((NN, DL), jnp.float32),
            pltpu.SemaphoreType.DMA,
        ],
    )
    def scatter(e_hbm, ridx_hbm, init_hbm, out_hbm, ridx_v, rows_v, agg_sh, sr):
        cid = lax.axis_index("c")
        sid = lax.axis_index("s")
        wid = sid * NC + cid
        base = wid * ept
        pltpu.sync_copy(ridx_hbm.at[wid], ridx_v)

        @pl.when(sid < NS - 1)
        def _():
            sl = pl.ds(sid * NPS, NPS)
            pltpu.sync_copy(init_hbm.at[cid, sl], agg_sh.at[sl])

        @pl.when(sid == NS - 1)
        def _():
            sl = pl.ds((NS - 1) * NPS, NPS_LAST)
            pltpu.sync_copy(init_hbm.at[cid, sl], agg_sh.at[sl])

        plsc.subcore_barrier()

        def start_read(j, b):
            pltpu.async_copy(e_hbm.at[pl.ds(base + j * chunk, chunk)],
                             rows_v.at[b], sr)

        def wait_read(j, b):
            pltpu.make_async_copy(e_hbm.at[pl.ds(base + j * chunk, chunk)],
                                  rows_v.at[b], sr).wait()

        start_read(0, 0)

        def body(j, carry):
            b = lax.rem(j, 2)

            @pl.when(j + 1 < nch)
            def _():
                start_read(j + 1, 1 - b)

            wait_read(j, b)
            pltpu.sync_copy(rows_v.at[b], agg_sh.at[ridx_v.at[j]], add=True)
            return carry

        lax.fori_loop(0, nch, body, 0)
        plsc.subcore_barrier()

        @pl.when(sid < NS - 1)
        def _():
            sl = pl.ds(sid * NPS, NPS)
            pltpu.sync_copy(agg_sh.at[sl], out_hbm.at[cid, sl])

        @pl.when(sid == NS - 1)
        def _():
            sl = pl.ds((NS - 1) * NPS, NPS_LAST)
            pltpu.sync_copy(agg_sh.at[sl], out_hbm.at[cid, sl])

    return scatter


NE_A = 62 * CHUNK * NW
NE_B = NE - NE_A
EPT_A, NCH_A = NE_A // NW, 62
EPT_B, NCH_B = NE_B // NW, 63
TILE_A = NE_A // 32
TILE_B = NE_B // 32

_gather_a = _make_gather(NE_A, EPT_A, CHUNK, NCH_A)
_gather_b = _make_gather(NE_B, EPT_B, CHUNK, NCH_B)
_scatter_a = _make_scatter(NE_A, EPT_A, CHUNK, NCH_A)
_scatter_b = _make_scatter(NE_B, EPT_B, CHUNK, NCH_B)


def kernel(nodes, edges, senders, receivers, params):
    nodes_p = jnp.pad(nodes, ((0, 0), (0, 3)))
    edges_p = jnp.pad(edges, ((0, 0), (0, 3)))
    pne = dict(params['node_enc'])
    pne['W1'] = jnp.pad(params['node_enc']['W1'], ((0, 3), (0, 0)))
    pee = dict(params['edge_enc'])
    pee['W1'] = jnp.pad(params['edge_enc']['W1'], ((0, 3), (0, 0)))

    pe = params['edge_mlp']
    w1e, w1s, w1r, w1ge = (pe['W1'][0:128], pe['W1'][128:256],
                           pe['W1'][256:384], pe['W1'][384:512])
    pn = params['node_mlp']
    w1n, w1a, w1gn = pn['W1'][0:128], pn['W1'][128:256], pn['W1'][256:384]
    pg = params['glob_mlp']
    wgn, wge, wgg = pg['W1'][0:128], pg['W1'][128:256], pg['W1'][256:384]

    sidx = [senders[:NE_A].reshape(NW, NCH_A, CHUNK),
            senders[NE_A:].reshape(NW, NCH_B, CHUNK)]
    ridx = [receivers[:NE_A].reshape(NW, NCH_A, CHUNK),
            receivers[NE_A:].reshape(NW, NCH_B, CHUNK)]
    agg_zero = jnp.zeros((NC, NN, DL), jnp.float32)

    node_lat = _encode(nodes_p, pne, tile=1000)
    elat = [_encode(edges_p[:NE_A], pee, tile=TILE_A),
            _encode(edges_p[NE_A:], pee, tile=TILE_B)]
    glob = jnp.zeros((1, DL), jnp.float32)

    for _ in range(4):
        ce, cn = _prep(glob, w1ge, pe['b1'], w1gn, pn['b1'])
        pt, qt = _pq(node_lat, w1s, w1r)
        gpA, gqA = _gather_a(pt, qt, sidx[0], ridx[0])
        gpB, gqB = _gather_b(pt, qt, sidx[1], ridx[1])
        elat[0], esA = _edge_step(elat[0], gpA, gqA, w1e, ce, pe, tile=TILE_A)
        aggA = _scatter_a(elat[0], ridx[0], agg_zero)
        elat[1], esB = _edge_step(elat[1], gpB, gqB, w1e, ce, pe, tile=TILE_B)
        agg2 = _scatter_b(elat[1], ridx[1], aggA)
        node_lat, nsum = _node_step(node_lat, agg2, w1n, w1a, cn, pn)
        glob = _glob_update(nsum, esA, esB, glob, wgn, wge, wgg, pg)

    out = _decode(glob, params['decoder'])
    return out * 1.0 + 0.0


# speedup vs baseline: 1.4542x; 1.0092x over previous
import functools

import jax
import jax.numpy as jnp
from jax import lax
from jax.experimental import pallas as pl
from jax.experimental.pallas import tpu as pltpu
from jax.experimental.pallas import tpu_sc as plsc

NN = 10000
NE = 320000
DL = 128

NC = 2
NS = 16
NW = NC * NS
EPT = NE // NW
CHUNK = 80
NCH = EPT // CHUNK
NPS = 624
NPS_LAST = NN - (NS - 1) * NPS

_mesh = plsc.VectorSubcoreMesh(core_axis_name="c", subcore_axis_name="s")


def _mlp_ln_body(x_ref, w1_ref, b1_ref, w2_ref, b2_ref, sc_ref, of_ref, o_ref):
    h = jnp.dot(x_ref[...], w1_ref[...], preferred_element_type=jnp.float32)
    h = jnp.maximum(h + b1_ref[...], 0.0)
    u = jnp.dot(h, w2_ref[...], preferred_element_type=jnp.float32) + b2_ref[...]
    mu = jnp.mean(u, axis=-1, keepdims=True)
    var = jnp.mean((u - mu) ** 2, axis=-1, keepdims=True)
    o_ref[...] = ((u - mu) * lax.rsqrt(var + 1e-5)) * sc_ref[...] + of_ref[...]


def _encode(x, p, tile):
    n, d = x.shape
    w1 = p['W1']
    full = lambda shape: pl.BlockSpec(shape, lambda i: (0, 0))
    return pl.pallas_call(
        _mlp_ln_body,
        grid=(n // tile,),
        in_specs=[
            pl.BlockSpec((tile, d), lambda i: (i, 0)),
            full((d, DL)), full((1, DL)), full((DL, DL)),
            full((1, DL)), full((1, DL)), full((1, DL)),
        ],
        out_specs=pl.BlockSpec((tile, DL), lambda i: (i, 0)),
        out_shape=jax.ShapeDtypeStruct((n, DL), jnp.float32),
    )(x, w1, p['b1'].reshape(1, -1), p['W2'], p['b2'].reshape(1, -1),
      p['scale'].reshape(1, -1), p['offset'].reshape(1, -1))


def _pq_body(x_ref, ws_ref, wr_ref, p_ref, q_ref):
    x = x_ref[...]
    p_ref[...] = jnp.dot(x, ws_ref[...], preferred_element_type=jnp.float32)
    q_ref[...] = jnp.dot(x, wr_ref[...], preferred_element_type=jnp.float32)


def _pq(node_lat, w1s, w1r, tile=1000):
    full = lambda shape: pl.BlockSpec(shape, lambda i: (0, 0))
    row = pl.BlockSpec((tile, DL), lambda i: (i, 0))
    return pl.pallas_call(
        _pq_body,
        grid=(NN // tile,),
        in_specs=[row, full((DL, DL)), full((DL, DL))],
        out_specs=[row, row],
        out_shape=[jax.ShapeDtypeStruct((NN, DL), jnp.float32)] * 2,
    )(node_lat, w1s, w1r)


def _prep_body(g_ref, wge_ref, b1e_ref, wgn_ref, b1n_ref, ce_ref, cn_ref):
    g = g_ref[...]
    ce_ref[...] = jnp.dot(g, wge_ref[...], preferred_element_type=jnp.float32) + b1e_ref[...]
    cn_ref[...] = jnp.dot(g, wgn_ref[...], preferred_element_type=jnp.float32) + b1n_ref[...]


def _prep(glob, wge, b1e, wgn, b1n):
    full = lambda shape: pl.BlockSpec(shape, lambda: (0, 0))
    return pl.pallas_call(
        _prep_body,
        in_specs=[full((1, DL)), full((DL, DL)), full((1, DL)),
                  full((DL, DL)), full((1, DL))],
        out_specs=[full((1, DL)), full((1, DL))],
        out_shape=[jax.ShapeDtypeStruct((1, DL), jnp.float32)] * 2,
    )(glob, wge, b1e.reshape(1, -1), wgn, b1n.reshape(1, -1))


def _edge_body(e_ref, gp_ref, gq_ref, w1_ref, c_ref, w2_ref, b2_ref,
               sc_ref, of_ref, o_ref, sum_ref):
    e = e_ref[...]
    h = jnp.dot(e, w1_ref[...], preferred_element_type=jnp.float32)
    h = jnp.maximum(h + gp_ref[...] + gq_ref[...] + c_ref[...], 0.0)
    u = jnp.dot(h, w2_ref[...], preferred_element_type=jnp.float32) + b2_ref[...]
    mu = jnp.mean(u, axis=-1, keepdims=True)
    var = jnp.mean((u - mu) ** 2, axis=-1, keepdims=True)
    new = e + ((u - mu) * lax.rsqrt(var + 1e-5)) * sc_ref[...] + of_ref[...]
    o_ref[...] = new

    @pl.when(pl.program_id(0) == 0)
    def _():
        sum_ref[...] = jnp.zeros_like(sum_ref)

    sum_ref[...] += jnp.sum(new, axis=0, keepdims=True)


def _edge_step(edge_lat, gp, gq, w1e, ce, p, tile=2000):
    ne = edge_lat.shape[0]
    full = lambda shape: pl.BlockSpec(shape, lambda i: (0, 0))
    row = pl.BlockSpec((tile, DL), lambda i: (i, 0))
    return pl.pallas_call(
        _edge_body,
        grid=(ne // tile,),
        in_specs=[row, row, row, full((DL, DL)), full((1, DL)),
                  full((DL, DL)), full((1, DL)), full((1, DL)), full((1, DL))],
        out_specs=[row, full((1, DL))],
        out_shape=[jax.ShapeDtypeStruct((ne, DL), jnp.float32),
                   jax.ShapeDtypeStruct((1, DL), jnp.float32)],
    )(edge_lat, gp, gq, w1e, ce, p['W2'], p['b2'].reshape(1, -1),
      p['scale'].reshape(1, -1), p['offset'].reshape(1, -1))


def _node_body(nl_ref, agg_ref, w1n_ref, w1a_ref, c_ref, w2_ref, b2_ref,
               sc_ref, of_ref, o_ref, sum_ref):
    nl = nl_ref[...]
    agg = agg_ref[0] + agg_ref[1]
    h = jnp.dot(nl, w1n_ref[...], preferred_element_type=jnp.float32)
    h = h + jnp.dot(agg, w1a_ref[...], preferred_element_type=jnp.float32)
    h = jnp.maximum(h + c_ref[...], 0.0)
    u = jnp.dot(h, w2_ref[...], preferred_element_type=jnp.float32) + b2_ref[...]
    mu = jnp.mean(u, axis=-1, keepdims=True)
    var = jnp.mean((u - mu) ** 2, axis=-1, keepdims=True)
    new = nl + ((u - mu) * lax.rsqrt(var + 1e-5)) * sc_ref[...] + of_ref[...]
    o_ref[...] = new

    @pl.when(pl.program_id(0) == 0)
    def _():
        sum_ref[...] = jnp.zeros_like(sum_ref)

    sum_ref[...] += jnp.sum(new, axis=0, keepdims=True)


def _node_step(node_lat, agg2, w1n, w1a, cn, p, tile=1000):
    full = lambda shape: pl.BlockSpec(shape, lambda i: (0, 0))
    row = pl.BlockSpec((tile, DL), lambda i: (i, 0))
    return pl.pallas_call(
        _node_body,
        grid=(NN // tile,),
        in_specs=[row, pl.BlockSpec((2, tile, DL), lambda i: (0, i, 0)),
                  full((DL, DL)), full((DL, DL)), full((1, DL)),
                  full((DL, DL)), full((1, DL)), full((1, DL)), full((1, DL))],
        out_specs=[row, full((1, DL))],
        out_shape=[jax.ShapeDtypeStruct((NN, DL), jnp.float32),
                   jax.ShapeDtypeStruct((1, DL), jnp.float32)],
    )(node_lat, agg2, w1n, w1a, cn, p['W2'], p['b2'].reshape(1, -1),
      p['scale'].reshape(1, -1), p['offset'].reshape(1, -1))


def _glob_body(ns_ref, es1_ref, es2_ref, g_ref, wgn_ref, wge_ref, wgg_ref,
               b1_ref, w2_ref, b2_ref, sc_ref, of_ref, o_ref):
    g = g_ref[...]
    es = es1_ref[...] + es2_ref[...]
    h = jnp.dot(ns_ref[...], wgn_ref[...], preferred_element_type=jnp.float32)
    h = h + jnp.dot(es, wge_ref[...], preferred_element_type=jnp.float32)
    h = h + jnp.dot(g, wgg_ref[...], preferred_element_type=jnp.float32)
    h = jnp.maximum(h + b1_ref[...], 0.0)
    u = jnp.dot(h, w2_ref[...], preferred_element_type=jnp.float32) + b2_ref[...]
    mu = jnp.mean(u, axis=-1, keepdims=True)
    var = jnp.mean((u - mu) ** 2, axis=-1, keepdims=True)
    o_ref[...] = g + ((u - mu) * lax.rsqrt(var + 1e-5)) * sc_ref[...] + of_ref[...]


def _glob_update(nsum, esum1, esum2, glob, wgn, wge, wgg, p):
    full = lambda shape: pl.BlockSpec(shape, lambda: (0, 0))
    return pl.pallas_call(
        _glob_body,
        in_specs=[full((1, DL))] * 4 + [full((DL, DL))] * 3 + [full((1, DL)),
                  full((DL, DL)), full((1, DL)), full((1, DL)), full((1, DL))],
        out_specs=full((1, DL)),
        out_shape=jax.ShapeDtypeStruct((1, DL), jnp.float32),
    )(nsum, esum1, esum2, glob, wgn, wge, wgg, p['b1'].reshape(1, -1), p['W2'],
      p['b2'].reshape(1, -1), p['scale'].reshape(1, -1), p['offset'].reshape(1, -1))


def _decode_body(g_ref, w1_ref, b1_ref, w2_ref, b2_ref, o_ref):
    h = jnp.dot(g_ref[...], w1_ref[...], preferred_element_type=jnp.float32)
    h = jnp.maximum(h + b1_ref[...], 0.0)
    o_ref[...] = jnp.dot(h, w2_ref[...], preferred_element_type=jnp.float32) + b2_ref[...]


def _decode(glob, p):
    full = lambda shape: pl.BlockSpec(shape, lambda: (0, 0))
    return pl.pallas_call(
        _decode_body,
        in_specs=[full((1, DL)), full((DL, DL)), full((1, DL)),
                  full((DL, 1)), full((1, 1))],
        out_specs=full((1, 1)),
        out_shape=jax.ShapeDtypeStruct((1, 1), jnp.float32),
    )(glob, p['W1'], p['b1'].reshape(1, -1), p['W2'], p['b2'].reshape(1, -1))


NBUF = 4


def _make_gather(ne, ept, chunk, nch):
    @functools.partial(
        pl.kernel,
        mesh=_mesh,
        out_type=[jax.ShapeDtypeStruct((ne, DL), jnp.float32),
                  jax.ShapeDtypeStruct((ne, DL), jnp.float32)],
        scratch_types=[
            pltpu.VMEM((nch, chunk), jnp.int32),
            pltpu.VMEM((nch, chunk), jnp.int32),
            pltpu.VMEM((NBUF, chunk, DL), jnp.float32),
            pltpu.VMEM((NBUF, chunk, DL), jnp.float32),
            pltpu.SemaphoreType.DMA,
            pltpu.SemaphoreType.DMA,
        ],
    )
    def gather(p_hbm, q_hbm, sidx_hbm, ridx_hbm, gp_hbm, gq_hbm,
               sidx_v, ridx_v, bp, bq, sg, sw):
        wid = lax.axis_index("s") * NC + lax.axis_index("c")
        base = wid * ept
        pltpu.sync_copy(sidx_hbm.at[wid], sidx_v)
        pltpu.sync_copy(ridx_hbm.at[wid], ridx_v)

        def start_gather(j, b):
            pltpu.async_copy(p_hbm.at[sidx_v.at[j]], bp.at[b], sg)
            pltpu.async_copy(q_hbm.at[ridx_v.at[j]], bq.at[b], sg)

        def wait_gather(j, b):
            pltpu.make_async_copy(p_hbm.at[sidx_v.at[j]], bp.at[b], sg).wait()
            pltpu.make_async_copy(q_hbm.at[ridx_v.at[j]], bq.at[b], sg).wait()

        def start_write(j, b):
            sl = pl.ds(base + j * chunk, chunk)
            pltpu.async_copy(bp.at[b], gp_hbm.at[sl], sw)
            pltpu.async_copy(bq.at[b], gq_hbm.at[sl], sw)

        def wait_write(j, b):
            sl = pl.ds(base + j * chunk, chunk)
            pltpu.make_async_copy(bp.at[b], gp_hbm.at[sl], sw).wait()
            pltpu.make_async_copy(bq.at[b], gq_hbm.at[sl], sw).wait()

        LOOK = 2
        for k in range(LOOK):
            start_gather(k, k)

        def body(j, carry):
            b = lax.rem(j, NBUF)

            @pl.when(j >= NBUF - LOOK)
            def _():
                wait_write(j - (NBUF - LOOK), lax.rem(j + LOOK, NBUF))

            @pl.when(j + LOOK < nch)
            def _():
                start_gather(j + LOOK, lax.rem(j + LOOK, NBUF))

            wait_gather(j, b)
            start_write(j, b)
            return carry

        lax.fori_loop(0, nch, body, 0)
        for k in range(NBUF - LOOK):
            j = nch - (NBUF - LOOK) + k
            wait_write(j, j % NBUF)

    return gather


def _make_scatter(ne, ept, chunk, nch):
    @functools.partial(
        pl.kernel,
        mesh=_mesh,
        out_type=jax.ShapeDtypeStruct((NC, NN, DL), jnp.float32),
        scratch_types=[
            pltpu.VMEM((nch, chunk), jnp.int32),
            pltpu.VMEM((2, chunk, DL), jnp.float32),
            pltpu.VMEM_SHARED((NN, DL), jnp.float32),
            pltpu.SemaphoreType.DMA,
        ],
    )
    def scatter(e_hbm, ridx_hbm, init_hbm, out_hbm, ridx_v, rows_v, agg_sh, sr):
        cid = lax.axis_index("c")
        sid = lax.axis_index("s")
        wid = sid * NC + cid
        base = wid * ept
        pltpu.sync_copy(ridx_hbm.at[wid], ridx_v)

        @pl.when(sid < NS - 1)
        def _():
            sl = pl.ds(sid * NPS, NPS)
            pltpu.sync_copy(init_hbm.at[cid, sl], agg_sh.at[sl])

        @pl.when(sid == NS - 1)
        def _():
            sl = pl.ds((NS - 1) * NPS, NPS_LAST)
            pltpu.sync_copy(init_hbm.at[cid, sl], agg_sh.at[sl])

        plsc.subcore_barrier()

        def start_read(j, b):
            pltpu.async_copy(e_hbm.at[pl.ds(base + j * chunk, chunk)],
                             rows_v.at[b], sr)

        def wait_read(j, b):
            pltpu.make_async_copy(e_hbm.at[pl.ds(base + j * chunk, chunk)],
                                  rows_v.at[b], sr).wait()

        start_read(0, 0)

        def body(j, carry):
            b = lax.rem(j, 2)

            @pl.when(j + 1 < nch)
            def _():
                start_read(j + 1, 1 - b)

            wait_read(j, b)
            pltpu.sync_copy(rows_v.at[b], agg_sh.at[ridx_v.at[j]], add=True)
            return carry

        lax.fori_loop(0, nch, body, 0)
        plsc.subcore_barrier()

        @pl.when(sid < NS - 1)
        def _():
            sl = pl.ds(sid * NPS, NPS)
            pltpu.sync_copy(agg_sh.at[sl], out_hbm.at[cid, sl])

        @pl.when(sid == NS - 1)
        def _():
            sl = pl.ds((NS - 1) * NPS, NPS_LAST)
            pltpu.sync_copy(agg_sh.at[sl], out_hbm.at[cid, sl])

    return scatter


NE_A = 62 * CHUNK * NW
NE_B = NE - NE_A
EPT_A, NCH_A = NE_A // NW, 62
EPT_B, NCH_B = NE_B // NW, 63
TILE_A = NE_A // 16
TILE_B = NE_B // 16

_gather_a = _make_gather(NE_A, EPT_A, CHUNK, NCH_A)
_gather_b = _make_gather(NE_B, EPT_B, CHUNK, NCH_B)
_scatter_a = _make_scatter(NE_A, EPT_A, CHUNK, NCH_A)
_scatter_b = _make_scatter(NE_B, EPT_B, CHUNK, NCH_B)


def kernel(nodes, edges, senders, receivers, params):
    nodes_p = jnp.pad(nodes, ((0, 0), (0, 3)))
    edges_p = jnp.pad(edges, ((0, 0), (0, 3)))
    pne = dict(params['node_enc'])
    pne['W1'] = jnp.pad(params['node_enc']['W1'], ((0, 3), (0, 0)))
    pee = dict(params['edge_enc'])
    pee['W1'] = jnp.pad(params['edge_enc']['W1'], ((0, 3), (0, 0)))

    pe = params['edge_mlp']
    w1e, w1s, w1r, w1ge = (pe['W1'][0:128], pe['W1'][128:256],
                           pe['W1'][256:384], pe['W1'][384:512])
    pn = params['node_mlp']
    w1n, w1a, w1gn = pn['W1'][0:128], pn['W1'][128:256], pn['W1'][256:384]
    pg = params['glob_mlp']
    wgn, wge, wgg = pg['W1'][0:128], pg['W1'][128:256], pg['W1'][256:384]

    sidx = [senders[:NE_A].reshape(NW, NCH_A, CHUNK),
            senders[NE_A:].reshape(NW, NCH_B, CHUNK)]
    ridx = [receivers[:NE_A].reshape(NW, NCH_A, CHUNK),
            receivers[NE_A:].reshape(NW, NCH_B, CHUNK)]
    agg_zero = jnp.zeros((NC, NN, DL), jnp.float32)

    node_lat = _encode(nodes_p, pne, tile=1000)
    elat = [_encode(edges_p[:NE_A], pee, tile=TILE_A),
            _encode(edges_p[NE_A:], pee, tile=TILE_B)]
    glob = jnp.zeros((1, DL), jnp.float32)

    for _ in range(4):
        ce, cn = _prep(glob, w1ge, pe['b1'], w1gn, pn['b1'])
        pt, qt = _pq(node_lat, w1s, w1r)
        gpA, gqA = _gather_a(pt, qt, sidx[0], ridx[0])
        gpB, gqB = _gather_b(pt, qt, sidx[1], ridx[1])
        elat[0], esA = _edge_step(elat[0], gpA, gqA, w1e, ce, pe, tile=TILE_A)
        aggA = _scatter_a(elat[0], ridx[0], agg_zero)
        elat[1], esB = _edge_step(elat[1], gpB, gqB, w1e, ce, pe, tile=TILE_B)
        agg2 = _scatter_b(elat[1], ridx[1], aggA)
        node_lat, nsum = _node_step(node_lat, agg2, w1n, w1a, cn, pn)
        glob = _glob_update(nsum, esA, esB, glob, wgn, wge, wgg, pg)

    out = _decode(glob, params['decoder'])
    return out * 1.0 + 0.0


# speedup vs baseline: 1.4675x; 1.0092x over previous
import functools

import jax
import jax.numpy as jnp
from jax import lax
from jax.experimental import pallas as pl
from jax.experimental.pallas import tpu as pltpu
from jax.experimental.pallas import tpu_sc as plsc

NN = 10000
NE = 320000
DL = 128

NC = 2
NS = 16
NW = NC * NS
EPT = NE // NW
CHUNK = 80
NCH = EPT // CHUNK
NPS = 624
NPS_LAST = NN - (NS - 1) * NPS

_mesh = plsc.VectorSubcoreMesh(core_axis_name="c", subcore_axis_name="s")


def _mlp_ln_body(x_ref, w1_ref, b1_ref, w2_ref, b2_ref, sc_ref, of_ref, o_ref):
    h = jnp.dot(x_ref[...], w1_ref[...], preferred_element_type=jnp.float32)
    h = jnp.maximum(h + b1_ref[...], 0.0)
    u = jnp.dot(h, w2_ref[...], preferred_element_type=jnp.float32) + b2_ref[...]
    mu = jnp.mean(u, axis=-1, keepdims=True)
    var = jnp.mean((u - mu) ** 2, axis=-1, keepdims=True)
    o_ref[...] = ((u - mu) * lax.rsqrt(var + 1e-5)) * sc_ref[...] + of_ref[...]


def _encode(x, p, tile):
    n, d = x.shape
    w1 = p['W1']
    full = lambda shape: pl.BlockSpec(shape, lambda i: (0, 0))
    return pl.pallas_call(
        _mlp_ln_body,
        grid=(n // tile,),
        in_specs=[
            pl.BlockSpec((tile, d), lambda i: (i, 0)),
            full((d, DL)), full((1, DL)), full((DL, DL)),
            full((1, DL)), full((1, DL)), full((1, DL)),
        ],
        out_specs=pl.BlockSpec((tile, DL), lambda i: (i, 0)),
        out_shape=jax.ShapeDtypeStruct((n, DL), jnp.float32),
    )(x, w1, p['b1'].reshape(1, -1), p['W2'], p['b2'].reshape(1, -1),
      p['scale'].reshape(1, -1), p['offset'].reshape(1, -1))


def _pq_body(x_ref, ws_ref, wr_ref, p_ref, q_ref):
    x = x_ref[...]
    p_ref[...] = jnp.dot(x, ws_ref[...], preferred_element_type=jnp.float32)
    q_ref[...] = jnp.dot(x, wr_ref[...], preferred_element_type=jnp.float32)


def _pq(node_lat, w1s, w1r, tile=2000):
    full = lambda shape: pl.BlockSpec(shape, lambda i: (0, 0))
    row = pl.BlockSpec((tile, DL), lambda i: (i, 0))
    return pl.pallas_call(
        _pq_body,
        grid=(NN // tile,),
        in_specs=[row, full((DL, DL)), full((DL, DL))],
        out_specs=[row, row],
        out_shape=[jax.ShapeDtypeStruct((NN, DL), jnp.float32)] * 2,
    )(node_lat, w1s, w1r)


def _prep_body(g_ref, wge_ref, b1e_ref, wgn_ref, b1n_ref, ce_ref, cn_ref):
    g = g_ref[...]
    ce_ref[...] = jnp.dot(g, wge_ref[...], preferred_element_type=jnp.float32) + b1e_ref[...]
    cn_ref[...] = jnp.dot(g, wgn_ref[...], preferred_element_type=jnp.float32) + b1n_ref[...]


def _prep(glob, wge, b1e, wgn, b1n):
    full = lambda shape: pl.BlockSpec(shape, lambda: (0, 0))
    return pl.pallas_call(
        _prep_body,
        in_specs=[full((1, DL)), full((DL, DL)), full((1, DL)),
                  full((DL, DL)), full((1, DL))],
        out_specs=[full((1, DL)), full((1, DL))],
        out_shape=[jax.ShapeDtypeStruct((1, DL), jnp.float32)] * 2,
    )(glob, wge, b1e.reshape(1, -1), wgn, b1n.reshape(1, -1))


def _edge_body(e_ref, gp_ref, gq_ref, w1_ref, c_ref, w2_ref, b2_ref,
               sc_ref, of_ref, o_ref, sum_ref):
    e = e_ref[...]
    h = jnp.dot(e, w1_ref[...], preferred_element_type=jnp.float32)
    h = jnp.maximum(h + gp_ref[...] + gq_ref[...] + c_ref[...], 0.0)
    u = jnp.dot(h, w2_ref[...], preferred_element_type=jnp.float32) + b2_ref[...]
    mu = jnp.mean(u, axis=-1, keepdims=True)
    var = jnp.mean((u - mu) ** 2, axis=-1, keepdims=True)
    new = e + ((u - mu) * lax.rsqrt(var + 1e-5)) * sc_ref[...] + of_ref[...]
    o_ref[...] = new

    @pl.when(pl.program_id(0) == 0)
    def _():
        sum_ref[...] = jnp.zeros_like(sum_ref)

    sum_ref[...] += jnp.sum(new, axis=0, keepdims=True)


def _edge_step(edge_lat, gp, gq, w1e, ce, p, tile=2000):
    ne = edge_lat.shape[0]
    full = lambda shape: pl.BlockSpec(shape, lambda i: (0, 0))
    row = pl.BlockSpec((tile, DL), lambda i: (i, 0))
    return pl.pallas_call(
        _edge_body,
        grid=(ne // tile,),
        in_specs=[row, row, row, full((DL, DL)), full((1, DL)),
                  full((DL, DL)), full((1, DL)), full((1, DL)), full((1, DL))],
        out_specs=[row, full((1, DL))],
        out_shape=[jax.ShapeDtypeStruct((ne, DL), jnp.float32),
                   jax.ShapeDtypeStruct((1, DL), jnp.float32)],
    )(edge_lat, gp, gq, w1e, ce, p['W2'], p['b2'].reshape(1, -1),
      p['scale'].reshape(1, -1), p['offset'].reshape(1, -1))


def _node_body(nl_ref, agg_ref, w1n_ref, w1a_ref, c_ref, w2_ref, b2_ref,
               sc_ref, of_ref, o_ref, sum_ref):
    nl = nl_ref[...]
    agg = agg_ref[0] + agg_ref[1]
    h = jnp.dot(nl, w1n_ref[...], preferred_element_type=jnp.float32)
    h = h + jnp.dot(agg, w1a_ref[...], preferred_element_type=jnp.float32)
    h = jnp.maximum(h + c_ref[...], 0.0)
    u = jnp.dot(h, w2_ref[...], preferred_element_type=jnp.float32) + b2_ref[...]
    mu = jnp.mean(u, axis=-1, keepdims=True)
    var = jnp.mean((u - mu) ** 2, axis=-1, keepdims=True)
    new = nl + ((u - mu) * lax.rsqrt(var + 1e-5)) * sc_ref[...] + of_ref[...]
    o_ref[...] = new

    @pl.when(pl.program_id(0) == 0)
    def _():
        sum_ref[...] = jnp.zeros_like(sum_ref)

    sum_ref[...] += jnp.sum(new, axis=0, keepdims=True)


def _node_step(node_lat, agg2, w1n, w1a, cn, p, tile=2000):
    full = lambda shape: pl.BlockSpec(shape, lambda i: (0, 0))
    row = pl.BlockSpec((tile, DL), lambda i: (i, 0))
    return pl.pallas_call(
        _node_body,
        grid=(NN // tile,),
        in_specs=[row, pl.BlockSpec((2, tile, DL), lambda i: (0, i, 0)),
                  full((DL, DL)), full((DL, DL)), full((1, DL)),
                  full((DL, DL)), full((1, DL)), full((1, DL)), full((1, DL))],
        out_specs=[row, full((1, DL))],
        out_shape=[jax.ShapeDtypeStruct((NN, DL), jnp.float32),
                   jax.ShapeDtypeStruct((1, DL), jnp.float32)],
    )(node_lat, agg2, w1n, w1a, cn, p['W2'], p['b2'].reshape(1, -1),
      p['scale'].reshape(1, -1), p['offset'].reshape(1, -1))


def _glob_body(ns_ref, es1_ref, es2_ref, g_ref, wgn_ref, wge_ref, wgg_ref,
               b1_ref, w2_ref, b2_ref, sc_ref, of_ref, o_ref):
    g = g_ref[...]
    es = es1_ref[...] + es2_ref[...]
    h = jnp.dot(ns_ref[...], wgn_ref[...], preferred_element_type=jnp.float32)
    h = h + jnp.dot(es, wge_ref[...], preferred_element_type=jnp.float32)
    h = h + jnp.dot(g, wgg_ref[...], preferred_element_type=jnp.float32)
    h = jnp.maximum(h + b1_ref[...], 0.0)
    u = jnp.dot(h, w2_ref[...], preferred_element_type=jnp.float32) + b2_ref[...]
    mu = jnp.mean(u, axis=-1, keepdims=True)
    var = jnp.mean((u - mu) ** 2, axis=-1, keepdims=True)
    o_ref[...] = g + ((u - mu) * lax.rsqrt(var + 1e-5)) * sc_ref[...] + of_ref[...]


def _glob_update(nsum, esum1, esum2, glob, wgn, wge, wgg, p):
    full = lambda shape: pl.BlockSpec(shape, lambda: (0, 0))
    return pl.pallas_call(
        _glob_body,
        in_specs=[full((1, DL))] * 4 + [full((DL, DL))] * 3 + [full((1, DL)),
                  full((DL, DL)), full((1, DL)), full((1, DL)), full((1, DL))],
        out_specs=full((1, DL)),
        out_shape=jax.ShapeDtypeStruct((1, DL), jnp.float32),
    )(nsum, esum1, esum2, glob, wgn, wge, wgg, p['b1'].reshape(1, -1), p['W2'],
      p['b2'].reshape(1, -1), p['scale'].reshape(1, -1), p['offset'].reshape(1, -1))


def _decode_body(g_ref, w1_ref, b1_ref, w2_ref, b2_ref, o_ref):
    h = jnp.dot(g_ref[...], w1_ref[...], preferred_element_type=jnp.float32)
    h = jnp.maximum(h + b1_ref[...], 0.0)
    o_ref[...] = jnp.dot(h, w2_ref[...], preferred_element_type=jnp.float32) + b2_ref[...]


def _decode(glob, p):
    full = lambda shape: pl.BlockSpec(shape, lambda: (0, 0))
    return pl.pallas_call(
        _decode_body,
        in_specs=[full((1, DL)), full((DL, DL)), full((1, DL)),
                  full((DL, 1)), full((1, 1))],
        out_specs=full((1, 1)),
        out_shape=jax.ShapeDtypeStruct((1, 1), jnp.float32),
    )(glob, p['W1'], p['b1'].reshape(1, -1), p['W2'], p['b2'].reshape(1, -1))


NBUF = 4


def _make_gather(ne, ept, chunk, nch):
    @functools.partial(
        pl.kernel,
        mesh=_mesh,
        out_type=[jax.ShapeDtypeStruct((ne, DL), jnp.float32),
                  jax.ShapeDtypeStruct((ne, DL), jnp.float32)],
        scratch_types=[
            pltpu.VMEM((nch, chunk), jnp.int32),
            pltpu.VMEM((nch, chunk), jnp.int32),
            pltpu.VMEM((NBUF, chunk, DL), jnp.float32),
            pltpu.VMEM((NBUF, chunk, DL), jnp.float32),
            pltpu.SemaphoreType.DMA,
            pltpu.SemaphoreType.DMA,
        ],
    )
    def gather(p_hbm, q_hbm, sidx_hbm, ridx_hbm, gp_hbm, gq_hbm,
               sidx_v, ridx_v, bp, bq, sg, sw):
        wid = lax.axis_index("s") * NC + lax.axis_index("c")
        base = wid * ept
        pltpu.sync_copy(sidx_hbm.at[wid], sidx_v)
        pltpu.sync_copy(ridx_hbm.at[wid], ridx_v)

        def start_gather(j, b):
            pltpu.async_copy(p_hbm.at[sidx_v.at[j]], bp.at[b], sg)
            pltpu.async_copy(q_hbm.at[ridx_v.at[j]], bq.at[b], sg)

        def wait_gather(j, b):
            pltpu.make_async_copy(p_hbm.at[sidx_v.at[j]], bp.at[b], sg).wait()
            pltpu.make_async_copy(q_hbm.at[ridx_v.at[j]], bq.at[b], sg).wait()

        def start_write(j, b):
            sl = pl.ds(base + j * chunk, chunk)
            pltpu.async_copy(bp.at[b], gp_hbm.at[sl], sw)
            pltpu.async_copy(bq.at[b], gq_hbm.at[sl], sw)

        def wait_write(j, b):
            sl = pl.ds(base + j * chunk, chunk)
            pltpu.make_async_copy(bp.at[b], gp_hbm.at[sl], sw).wait()
            pltpu.make_async_copy(bq.at[b], gq_hbm.at[sl], sw).wait()

        LOOK = 2
        for k in range(LOOK):
            start_gather(k, k)

        def body(j, carry):
            b = lax.rem(j, NBUF)

            @pl.when(j >= NBUF - LOOK)
            def _():
                wait_write(j - (NBUF - LOOK), lax.rem(j + LOOK, NBUF))

            @pl.when(j + LOOK < nch)
            def _():
                start_gather(j + LOOK, lax.rem(j + LOOK, NBUF))

            wait_gather(j, b)
            start_write(j, b)
            return carry

        lax.fori_loop(0, nch, body, 0)
        for k in range(NBUF - LOOK):
            j = nch - (NBUF - LOOK) + k
            wait_write(j, j % NBUF)

    return gather


def _make_scatter(ne, ept, chunk, nch):
    @functools.partial(
        pl.kernel,
        mesh=_mesh,
        out_type=jax.ShapeDtypeStruct((NC, NN, DL), jnp.float32),
        scratch_types=[
            pltpu.VMEM((nch, chunk), jnp.int32),
            pltpu.VMEM((2, chunk, DL), jnp.float32),
            pltpu.VMEM_SHARED((NN, DL), jnp.float32),
            pltpu.SemaphoreType.DMA,
        ],
    )
    def scatter(e_hbm, ridx_hbm, init_hbm, out_hbm, ridx_v, rows_v, agg_sh, sr):
        cid = lax.axis_index("c")
        sid = lax.axis_index("s")
        wid = sid * NC + cid
        base = wid * ept
        pltpu.sync_copy(ridx_hbm.at[wid], ridx_v)

        @pl.when(sid < NS - 1)
        def _():
            sl = pl.ds(sid * NPS, NPS)
            pltpu.sync_copy(init_hbm.at[cid, sl], agg_sh.at[sl])

        @pl.when(sid == NS - 1)
        def _():
            sl = pl.ds((NS - 1) * NPS, NPS_LAST)
            pltpu.sync_copy(init_hbm.at[cid, sl], agg_sh.at[sl])

        plsc.subcore_barrier()

        def start_read(j, b):
            pltpu.async_copy(e_hbm.at[pl.ds(base + j * chunk, chunk)],
                             rows_v.at[b], sr)

        def wait_read(j, b):
            pltpu.make_async_copy(e_hbm.at[pl.ds(base + j * chunk, chunk)],
                                  rows_v.at[b], sr).wait()

        start_read(0, 0)

        def body(j, carry):
            b = lax.rem(j, 2)

            @pl.when(j + 1 < nch)
            def _():
                start_read(j + 1, 1 - b)

            wait_read(j, b)
            pltpu.sync_copy(rows_v.at[b], agg_sh.at[ridx_v.at[j]], add=True)
            return carry

        lax.fori_loop(0, nch, body, 0)
        plsc.subcore_barrier()

        @pl.when(sid < NS - 1)
        def _():
            sl = pl.ds(sid * NPS, NPS)
            pltpu.sync_copy(agg_sh.at[sl], out_hbm.at[cid, sl])

        @pl.when(sid == NS - 1)
        def _():
            sl = pl.ds((NS - 1) * NPS, NPS_LAST)
            pltpu.sync_copy(agg_sh.at[sl], out_hbm.at[cid, sl])

    return scatter


NE_A = 62 * CHUNK * NW
NE_B = NE - NE_A
EPT_A, NCH_A = NE_A // NW, 62
EPT_B, NCH_B = NE_B // NW, 63
TILE_A = NE_A // 16
TILE_B = NE_B // 16

_gather_a = _make_gather(NE_A, EPT_A, CHUNK, NCH_A)
_gather_b = _make_gather(NE_B, EPT_B, CHUNK, NCH_B)
_scatter_a = _make_scatter(NE_A, EPT_A, CHUNK, NCH_A)
_scatter_b = _make_scatter(NE_B, EPT_B, CHUNK, NCH_B)


def kernel(nodes, edges, senders, receivers, params):
    nodes_p = jnp.pad(nodes, ((0, 0), (0, 3)))
    edges_p = jnp.pad(edges, ((0, 0), (0, 3)))
    pne = dict(params['node_enc'])
    pne['W1'] = jnp.pad(params['node_enc']['W1'], ((0, 3), (0, 0)))
    pee = dict(params['edge_enc'])
    pee['W1'] = jnp.pad(params['edge_enc']['W1'], ((0, 3), (0, 0)))

    pe = params['edge_mlp']
    w1e, w1s, w1r, w1ge = (pe['W1'][0:128], pe['W1'][128:256],
                           pe['W1'][256:384], pe['W1'][384:512])
    pn = params['node_mlp']
    w1n, w1a, w1gn = pn['W1'][0:128], pn['W1'][128:256], pn['W1'][256:384]
    pg = params['glob_mlp']
    wgn, wge, wgg = pg['W1'][0:128], pg['W1'][128:256], pg['W1'][256:384]

    sidx = [senders[:NE_A].reshape(NW, NCH_A, CHUNK),
            senders[NE_A:].reshape(NW, NCH_B, CHUNK)]
    ridx = [receivers[:NE_A].reshape(NW, NCH_A, CHUNK),
            receivers[NE_A:].reshape(NW, NCH_B, CHUNK)]
    agg_zero = jnp.zeros((NC, NN, DL), jnp.float32)

    node_lat = _encode(nodes_p, pne, tile=2000)
    elat = [_encode(edges_p[:NE_A], pee, tile=TILE_A),
            _encode(edges_p[NE_A:], pee, tile=TILE_B)]
    glob = jnp.zeros((1, DL), jnp.float32)

    for _ in range(4):
        ce, cn = _prep(glob, w1ge, pe['b1'], w1gn, pn['b1'])
        pt, qt = _pq(node_lat, w1s, w1r)
        gpA, gqA = _gather_a(pt, qt, sidx[0], ridx[0])
        gpB, gqB = _gather_b(pt, qt, sidx[1], ridx[1])
        elat[0], esA = _edge_step(elat[0], gpA, gqA, w1e, ce, pe, tile=TILE_A)
        aggA = _scatter_a(elat[0], ridx[0], agg_zero)
        elat[1], esB = _edge_step(elat[1], gpB, gqB, w1e, ce, pe, tile=TILE_B)
        agg2 = _scatter_b(elat[1], ridx[1], aggA)
        node_lat, nsum = _node_step(node_lat, agg2, w1n, w1a, cn, pn)
        glob = _glob_update(nsum, esA, esB, glob, wgn, wge, wgg, pg)

    out = _decode(glob, params['decoder'])
    return out * 1.0 + 0.0
